# Initial kernel scaffold; baseline (speedup 1.0000x reference)
import jax
import jax.numpy as jnp
from jax.experimental import pallas as pl


def kernel(mod0, mod1, mod2, p0_w, p0_b, p2_w, p2_b, ln_g, ln_b, gate_w, gate_b, e_w1, e_b1, e_w2, e_b2):
    raise NotImplementedError("write your pallas kernel here")


# speedup vs baseline: 1.5613x; 1.5613x over previous
import functools

import jax
import jax.numpy as jnp
from jax import lax
from jax.experimental import pallas as pl
from jax.experimental.pallas import tpu as pltpu

B = 1024
D = 1024
M = 3
E = 8
H = 2048
TB = 128
NBT = B // TB
CAP = B
CTILE = CAP // TB


def _erf(x):
    a1, a2, a3, a4, a5 = (
        0.254829592, -0.284496736, 1.421413741, -1.453152027, 1.061405429)
    p = 0.3275911
    s = jnp.sign(x)
    ax = jnp.abs(x)
    t = 1.0 / (1.0 + p * ax)
    poly = t * (a1 + t * (a2 + t * (a3 + t * (a4 + t * a5))))
    y = 1.0 - poly * jnp.exp(-ax * ax)
    return s * y


def _gelu(x):
    return 0.5 * x * (1.0 + _erf(x * 0.7071067811865476))


def _k1_body(mod0_ref, mod1_ref, mod2_ref, p0w_ref, p0b_ref, p2w_ref,
             p2b_ref, lng_ref, lnb_ref, gw_ref, gb_ref,
             proj_ref, tok_ref, gates_ref, slot0_ref, slot1_ref,
             g0_ref, g1_ref, counts_ref, valid_ref, bloss_ref,
             base_ref, imp_ref):
    bt = pl.program_id(0)
    bf = jnp.bfloat16

    def bdot(a, b):
        return lax.dot_general(a.astype(bf), b.astype(bf),
                               (((1,), (1,)), ((), ())),
                               preferred_element_type=jnp.float32)

    t0 = bdot(mod0_ref[...], p0w_ref[...]) + p0b_ref[...]
    t1 = mod1_ref[...]
    t2 = bdot(mod2_ref[...], p2w_ref[...]) + p2b_ref[...]

    proj_ref[:, 0, :] = t0
    proj_ref[:, 1, :] = t1
    proj_ref[:, 2, :] = t2

    def ln(t):
        mu = jnp.mean(t, axis=-1, keepdims=True)
        var = jnp.mean((t - mu) ** 2, axis=-1, keepdims=True)
        return (t - mu) / jnp.sqrt(var + 1e-5) * lng_ref[...] + lnb_ref[...]

    n0, n1, n2 = ln(t0), ln(t1), ln(t2)
    tok_ref[:, 0, :] = n0
    tok_ref[:, 1, :] = n1
    tok_ref[:, 2, :] = n2

    ctx = jnp.concatenate([n0, n1, n2], axis=-1)
    logits = bdot(ctx, gw_ref[...]) + gb_ref[...]

    eidx = lax.broadcasted_iota(jnp.int32, (TB, E), 1)
    m1 = jnp.max(logits, axis=-1, keepdims=True)
    idx1 = jnp.min(jnp.where(logits == m1, eidx, E), axis=-1, keepdims=True)
    oh1 = eidx == idx1
    masked = jnp.where(oh1, -jnp.inf, logits)
    m2 = jnp.max(masked, axis=-1, keepdims=True)
    idx2 = jnp.min(jnp.where(masked == m2, eidx, E), axis=-1, keepdims=True)
    oh2 = eidx == idx2

    ex = jnp.exp(m2 - m1)
    den = 1.0 + ex
    g1v = 1.0 / den
    g2v = ex / den
    gates = jnp.where(oh1, g1v, jnp.where(oh2, g2v, 0.0))
    gates_ref[...] = gates

    @pl.when(bt == 0)
    def _():
        base_ref[...] = jnp.zeros_like(base_ref)
        imp_ref[...] = jnp.zeros_like(imp_ref)

    mask = (oh1 | oh2).astype(jnp.float32)
    bi = lax.broadcasted_iota(jnp.int32, (TB, TB), 0)
    bj = lax.broadcasted_iota(jnp.int32, (TB, TB), 1)
    tri = (bj < bi).astype(jnp.bfloat16)
    pos = lax.dot_general(tri, mask.astype(jnp.bfloat16),
                          (((1,), (0,)), ((), ())),
                          preferred_element_type=jnp.float32)
    posg = pos + base_ref[...]

    pos1 = jnp.sum(jnp.where(oh1, posg, 0.0), axis=-1).astype(jnp.int32)
    pos2 = jnp.sum(jnp.where(oh2, posg, 0.0), axis=-1).astype(jnp.int32)
    slot0_ref[...] = (idx1[:, 0] * CAP + pos1)[None, None, :]
    slot1_ref[...] = (idx2[:, 0] * CAP + pos2)[None, None, :]
    g0_ref[...] = g1v[:, 0][None, None, :]
    g1_ref[...] = g2v[:, 0][None, None, :]

    base_ref[...] = base_ref[...] + jnp.sum(mask, axis=0, keepdims=True)
    imp_ref[...] = imp_ref[...] + jnp.sum(gates, axis=0, keepdims=True)

    @pl.when(bt == NBT - 1)
    def _():
        cnt = base_ref[...].astype(jnp.int32)
        counts_ref[...] = cnt
        jt = lax.broadcasted_iota(jnp.int32, (E, CTILE), 1) * TB
        valid_ref[...] = (jt < cnt.reshape(E, 1)).astype(jnp.int32)
        imp = imp_ref[...]
        mu = jnp.mean(imp)
        var = jnp.mean((imp - mu) ** 2)
        bloss_ref[...] = jnp.reshape(0.01 * var / (mu * mu + 1e-10), (1, 1))


def _k1(mod0, mod1, mod2, p0_w, p0_b, p2_w, p2_b, ln_g, ln_b, gate_w, gate_b):
    full = lambda s: pl.BlockSpec(s, lambda bt: (0,) * len(s))
    outs = (
        jax.ShapeDtypeStruct((B, M, D), jnp.float32),
        jax.ShapeDtypeStruct((B, M, D), jnp.float32),
        jax.ShapeDtypeStruct((B, E), jnp.float32),
        jax.ShapeDtypeStruct((NBT, 1, TB), jnp.int32),
        jax.ShapeDtypeStruct((NBT, 1, TB), jnp.int32),
        jax.ShapeDtypeStruct((NBT, 1, TB), jnp.float32),
        jax.ShapeDtypeStruct((NBT, 1, TB), jnp.float32),
        jax.ShapeDtypeStruct((1, E), jnp.int32),
        jax.ShapeDtypeStruct((E, CTILE), jnp.int32),
        jax.ShapeDtypeStruct((1, 1), jnp.float32),
    )
    return pl.pallas_call(
        _k1_body,
        grid=(NBT,),
        in_specs=[
            pl.BlockSpec((TB, 768), lambda bt: (bt, 0)),
            pl.BlockSpec((TB, D), lambda bt: (bt, 0)),
            pl.BlockSpec((TB, 512), lambda bt: (bt, 0)),
            full((D, 768)),
            full((1, D)),
            full((D, 512)),
            full((1, D)),
            full((1, D)),
            full((1, D)),
            full((E, M * D)),
            full((1, E)),
        ],
        out_specs=(
            pl.BlockSpec((TB, M, D), lambda bt: (bt, 0, 0)),
            pl.BlockSpec((TB, M, D), lambda bt: (bt, 0, 0)),
            pl.BlockSpec((TB, E), lambda bt: (bt, 0)),
            pl.BlockSpec((1, 1, TB), lambda bt: (bt, 0, 0)),
            pl.BlockSpec((1, 1, TB), lambda bt: (bt, 0, 0)),
            pl.BlockSpec((1, 1, TB), lambda bt: (bt, 0, 0)),
            pl.BlockSpec((1, 1, TB), lambda bt: (bt, 0, 0)),
            full((1, E)),
            full((E, CTILE)),
            full((1, 1)),
        ),
        out_shape=outs,
        scratch_shapes=[
            pltpu.VMEM((1, E), jnp.float32),
            pltpu.VMEM((1, E), jnp.float32),
        ],
    )(mod0, mod1, mod2, p0_w, p0_b.reshape(1, D), p2_w, p2_b.reshape(1, D),
      ln_g.reshape(1, D), ln_b.reshape(1, D), gate_w, gate_b.reshape(1, E))


def _dense_moe_body(tok_ref, w1_ref, b1_ref, w2_ref, b2_ref, gates_ref,
                    out_ref, pooled_ref):
    e = pl.program_id(1)
    x = tok_ref[...].astype(jnp.bfloat16)
    w1 = w1_ref[0].astype(jnp.bfloat16)
    h = jnp.dot(x, w1, preferred_element_type=jnp.float32) + b1_ref[0]
    h = _gelu(h)
    w2 = w2_ref[0].astype(jnp.bfloat16)
    y = (jnp.dot(h.astype(jnp.bfloat16), w2,
                 preferred_element_type=jnp.float32) + b2_ref[0])
    eoh = (lax.broadcasted_iota(jnp.int32, (E, 1), 0) == e).astype(jnp.float32)
    gcol = jnp.dot(gates_ref[...], eoh)
    gexp = jnp.repeat(gcol, M, axis=0)
    contrib = y * gexp

    @pl.when(e == 0)
    def _():
        out_ref[...] = contrib

    @pl.when(e != 0)
    def _():
        out_ref[...] = out_ref[...] + contrib

    @pl.when(e == E - 1)
    def _():
        o3 = out_ref[...].reshape(TB, M, D)
        pooled_ref[...] = (o3[:, 0, :] + o3[:, 1, :] + o3[:, 2, :]) * (1.0 / M)


def _dense_moe(tokens_flat, e_w1, e_b1, e_w2, e_b2, gates):
    outs = (
        jax.ShapeDtypeStruct((B * M, D), jnp.float32),
        jax.ShapeDtypeStruct((B, D), jnp.float32),
    )
    return pl.pallas_call(
        _dense_moe_body,
        grid=(NBT, E),
        in_specs=[
            pl.BlockSpec((M * TB, D), lambda bt, e: (bt, 0)),
            pl.BlockSpec((1, D, H), lambda bt, e: (e, 0, 0)),
            pl.BlockSpec((1, 1, H), lambda bt, e: (e, 0, 0)),
            pl.BlockSpec((1, H, D), lambda bt, e: (e, 0, 0)),
            pl.BlockSpec((1, 1, D), lambda bt, e: (e, 0, 0)),
            pl.BlockSpec((TB, E), lambda bt, e: (bt, 0)),
        ],
        out_specs=(
            pl.BlockSpec((M * TB, D), lambda bt, e: (bt, 0)),
            pl.BlockSpec((TB, D), lambda bt, e: (bt, 0)),
        ),
        out_shape=outs,
    )(tokens_flat, e_w1, e_b1.reshape(E, 1, H), e_w2, e_b2.reshape(E, 1, D),
      gates)


def kernel(mod0, mod1, mod2, p0_w, p0_b, p2_w, p2_b, ln_g, ln_b, gate_w,
           gate_b, e_w1, e_b1, e_w2, e_b2):
    (projected, tokens, gates, slot0, slot1, g0, g1, counts, valid,
     bloss) = _k1(mod0, mod1, mod2, p0_w, p0_b, p2_w, p2_b, ln_g, ln_b,
                  gate_w, gate_b)
    tokens_flat = tokens.reshape(B * M, D)
    out_flat, pooled = _dense_moe(tokens_flat, e_w1, e_b1, e_w2, e_b2, gates)
    modality_tokens = out_flat.reshape(B, M, D)
    return pooled, modality_tokens, projected, gates, bloss[0, 0]


# speedup vs baseline: 2.3509x; 1.5057x over previous
import jax
import jax.numpy as jnp
from jax import lax
from jax.experimental import pallas as pl
from jax.experimental.pallas import tpu as pltpu
from jax.experimental.pallas import tpu_sc as plsc

B = 1024
D = 1024
M = 3
E = 8
H = 2048
TB = 128
NBT = B // TB
CAP = B
CTILE = CAP // TB
NT = E * CTILE
SROWS = TB * M
NSLOT3 = E * CAP * M

NC = 2
NS = 16
NW = NC * NS
SPW = B // NW
WROWS = SPW * M


def _erf(x):
    a1, a2, a3, a4, a5 = (
        0.254829592, -0.284496736, 1.421413741, -1.453152027, 1.061405429)
    p = 0.3275911
    s = jnp.sign(x)
    ax = jnp.abs(x)
    t = 1.0 / (1.0 + p * ax)
    poly = t * (a1 + t * (a2 + t * (a3 + t * (a4 + t * a5))))
    y = 1.0 - poly * jnp.exp(-ax * ax)
    return s * y


def _gelu(x):
    return 0.5 * x * (1.0 + _erf(x * 0.7071067811865476))


def _k1_body(mod0_ref, mod1_ref, mod2_ref, p0w_ref, p0b_ref, p2w_ref,
             p2b_ref, lng_ref, lnb_ref, gw_ref, gb_ref,
             proj_ref, tok_ref, gates_ref, slot0_ref, slot1_ref,
             g0_ref, g1_ref, valid_ref, bloss_ref,
             base_ref, imp_ref):
    bt = pl.program_id(0)
    bf = jnp.bfloat16

    def bdot(a, b):
        return lax.dot_general(a.astype(bf), b.astype(bf),
                               (((1,), (1,)), ((), ())),
                               preferred_element_type=jnp.float32)

    t0 = bdot(mod0_ref[...], p0w_ref[...]) + p0b_ref[...]
    t1 = mod1_ref[...]
    t2 = bdot(mod2_ref[...], p2w_ref[...]) + p2b_ref[...]

    proj_ref[:, 0, :] = t0
    proj_ref[:, 1, :] = t1
    proj_ref[:, 2, :] = t2

    def ln(t):
        mu = jnp.mean(t, axis=-1, keepdims=True)
        var = jnp.mean((t - mu) ** 2, axis=-1, keepdims=True)
        return (t - mu) / jnp.sqrt(var + 1e-5) * lng_ref[...] + lnb_ref[...]

    n0, n1, n2 = ln(t0), ln(t1), ln(t2)
    tok_ref[:, 0, :] = n0
    tok_ref[:, 1, :] = n1
    tok_ref[:, 2, :] = n2

    ctx = jnp.concatenate([n0, n1, n2], axis=-1)
    logits = bdot(ctx, gw_ref[...]) + gb_ref[...]

    eidx = lax.broadcasted_iota(jnp.int32, (TB, E), 1)
    m1 = jnp.max(logits, axis=-1, keepdims=True)
    idx1 = jnp.min(jnp.where(logits == m1, eidx, E), axis=-1, keepdims=True)
    oh1 = eidx == idx1
    masked = jnp.where(oh1, -jnp.inf, logits)
    m2 = jnp.max(masked, axis=-1, keepdims=True)
    idx2 = jnp.min(jnp.where(masked == m2, eidx, E), axis=-1, keepdims=True)
    oh2 = eidx == idx2

    ex = jnp.exp(m2 - m1)
    den = 1.0 + ex
    g1v = 1.0 / den
    g2v = ex / den
    gates = jnp.where(oh1, g1v, jnp.where(oh2, g2v, 0.0))
    gates_ref[...] = gates

    @pl.when(bt == 0)
    def _():
        base_ref[...] = jnp.zeros_like(base_ref)
        imp_ref[...] = jnp.zeros_like(imp_ref)

    mask = (oh1 | oh2).astype(jnp.float32)
    bi = lax.broadcasted_iota(jnp.int32, (TB, TB), 0)
    bj = lax.broadcasted_iota(jnp.int32, (TB, TB), 1)
    tri = (bj < bi).astype(jnp.bfloat16)
    pos = lax.dot_general(tri, mask.astype(jnp.bfloat16),
                          (((1,), (0,)), ((), ())),
                          preferred_element_type=jnp.float32)
    posg = pos + base_ref[...]

    pos1 = jnp.sum(jnp.where(oh1, posg, 0.0), axis=-1).astype(jnp.int32)
    pos2 = jnp.sum(jnp.where(oh2, posg, 0.0), axis=-1).astype(jnp.int32)
    slot0_ref[...] = (idx1[:, 0] * CAP + pos1)[:, None]
    slot1_ref[...] = (idx2[:, 0] * CAP + pos2)[:, None]
    g0_ref[...] = g1v
    g1_ref[...] = g2v

    base_ref[...] = base_ref[...] + jnp.sum(mask, axis=0, keepdims=True)
    imp_ref[...] = imp_ref[...] + jnp.sum(gates, axis=0, keepdims=True)

    @pl.when(bt == NBT - 1)
    def _():
        cnt = base_ref[...].astype(jnp.int32)
        jt = lax.broadcasted_iota(jnp.int32, (E, CTILE), 1) * TB
        valid_ref[...] = (jt < cnt.reshape(E, 1)).astype(jnp.int32)
        imp = imp_ref[...]
        mu = jnp.mean(imp)
        var = jnp.mean((imp - mu) ** 2)
        bloss_ref[...] = jnp.reshape(0.01 * var / (mu * mu + 1e-10), (1, 1))


def _k1(mod0, mod1, mod2, p0_w, p0_b, p2_w, p2_b, ln_g, ln_b, gate_w, gate_b):
    full = lambda s: pl.BlockSpec(s, lambda bt: (0,) * len(s))
    outs = (
        jax.ShapeDtypeStruct((B, M, D), jnp.float32),
        jax.ShapeDtypeStruct((B, M, D), jnp.float32),
        jax.ShapeDtypeStruct((B, E), jnp.float32),
        jax.ShapeDtypeStruct((B, 1), jnp.int32),
        jax.ShapeDtypeStruct((B, 1), jnp.int32),
        jax.ShapeDtypeStruct((B, 1), jnp.float32),
        jax.ShapeDtypeStruct((B, 1), jnp.float32),
        jax.ShapeDtypeStruct((E, CTILE), jnp.int32),
        jax.ShapeDtypeStruct((1, 1), jnp.float32),
    )
    return pl.pallas_call(
        _k1_body,
        grid=(NBT,),
        in_specs=[
            pl.BlockSpec((TB, 768), lambda bt: (bt, 0)),
            pl.BlockSpec((TB, D), lambda bt: (bt, 0)),
            pl.BlockSpec((TB, 512), lambda bt: (bt, 0)),
            full((D, 768)),
            full((1, D)),
            full((D, 512)),
            full((1, D)),
            full((1, D)),
            full((1, D)),
            full((E, M * D)),
            full((1, E)),
        ],
        out_specs=(
            pl.BlockSpec((TB, M, D), lambda bt: (bt, 0, 0)),
            pl.BlockSpec((TB, M, D), lambda bt: (bt, 0, 0)),
            pl.BlockSpec((TB, E), lambda bt: (bt, 0)),
            pl.BlockSpec((TB, 1), lambda bt: (bt, 0)),
            pl.BlockSpec((TB, 1), lambda bt: (bt, 0)),
            pl.BlockSpec((TB, 1), lambda bt: (bt, 0)),
            pl.BlockSpec((TB, 1), lambda bt: (bt, 0)),
            full((E, CTILE)),
            full((1, 1)),
        ),
        out_shape=outs,
        scratch_shapes=[
            pltpu.VMEM((1, E), jnp.float32),
            pltpu.VMEM((1, E), jnp.float32),
        ],
    )(mod0, mod1, mod2, p0_w, p0_b.reshape(1, D), p2_w, p2_b.reshape(1, D),
      ln_g.reshape(1, D), ln_b.reshape(1, D), gate_w, gate_b.reshape(1, E))


def _build_idx(idx_ref, sl_ref):
    it = lax.broadcasted_iota(jnp.int32, (16,), 0)
    for g in range(SPW // 16):
        s = sl_ref[pl.ds(g * 16, 16)]
        for r in range(M):
            plsc.store_scatter(idx_ref, [it * 3 + (g * 48 + r)], s * 3 + r)


def _sc_scatter_body(tok_hbm, s0_hbm, s1_hbm, disp_hbm, buf, idx, sl, sem):
    wid = lax.axis_index("s") * NC + lax.axis_index("c")
    base = wid * SPW
    pltpu.sync_copy(tok_hbm.at[pl.ds(base * M, WROWS)], buf)
    pltpu.sync_copy(s0_hbm.at[pl.ds(base, SPW)], sl)
    _build_idx(idx, sl)
    pltpu.async_copy(buf, disp_hbm.at[idx], sem).wait()
    pltpu.sync_copy(s1_hbm.at[pl.ds(base, SPW)], sl)
    _build_idx(idx, sl)
    pltpu.async_copy(buf, disp_hbm.at[idx], sem).wait()


def _sc_scatter(tokens_flat, s0, s1):
    mesh = plsc.VectorSubcoreMesh(core_axis_name="c", subcore_axis_name="s")
    return pl.kernel(
        _sc_scatter_body,
        out_type=jax.ShapeDtypeStruct((NSLOT3, D), jnp.float32),
        mesh=mesh,
        compiler_params=pltpu.CompilerParams(needs_layout_passes=False),
        scratch_types=[
            pltpu.VMEM((WROWS, D), jnp.float32),
            pltpu.VMEM((WROWS,), jnp.int32),
            pltpu.VMEM((SPW,), jnp.int32),
            pltpu.SemaphoreType.DMA,
        ],
    )(tokens_flat, s0, s1)


def _sc_gather_body(y_hbm, s0_hbm, s1_hbm, y0_hbm, y1_hbm, buf, idx, sl, sem):
    wid = lax.axis_index("s") * NC + lax.axis_index("c")
    base = wid * SPW
    pltpu.sync_copy(s0_hbm.at[pl.ds(base, SPW)], sl)
    _build_idx(idx, sl)
    pltpu.async_copy(y_hbm.at[idx], buf, sem).wait()
    pltpu.sync_copy(buf, y0_hbm.at[pl.ds(base * M, WROWS)])
    pltpu.sync_copy(s1_hbm.at[pl.ds(base, SPW)], sl)
    _build_idx(idx, sl)
    pltpu.async_copy(y_hbm.at[idx], buf, sem).wait()
    pltpu.sync_copy(buf, y1_hbm.at[pl.ds(base * M, WROWS)])


def _sc_gather(y, s0, s1):
    mesh = plsc.VectorSubcoreMesh(core_axis_name="c", subcore_axis_name="s")
    return pl.kernel(
        _sc_gather_body,
        out_type=(jax.ShapeDtypeStruct((B * M, D), jnp.float32),
                  jax.ShapeDtypeStruct((B * M, D), jnp.float32)),
        mesh=mesh,
        compiler_params=pltpu.CompilerParams(needs_layout_passes=False),
        scratch_types=[
            pltpu.VMEM((WROWS, D), jnp.float32),
            pltpu.VMEM((WROWS,), jnp.int32),
            pltpu.VMEM((SPW,), jnp.int32),
            pltpu.SemaphoreType.DMA,
        ],
    )(y, s0, s1)


def _moe_body(valid_ref, x_ref, w1_ref, b1_ref, w2_ref, b2_ref, y_ref):
    t = pl.program_id(0)

    @pl.when(valid_ref[t] == 1)
    def _():
        x = x_ref[...].astype(jnp.bfloat16)
        w1 = w1_ref[0].astype(jnp.bfloat16)
        h = jnp.dot(x, w1, preferred_element_type=jnp.float32) + b1_ref[0]
        h = _gelu(h)
        w2 = w2_ref[0].astype(jnp.bfloat16)
        y_ref[...] = (jnp.dot(h.astype(jnp.bfloat16), w2,
                              preferred_element_type=jnp.float32) + b2_ref[0])


def _sparse_moe(valid_flat, disp, e_w1, e_b1, e_w2, e_b2):
    grid_spec = pltpu.PrefetchScalarGridSpec(
        num_scalar_prefetch=1,
        grid=(NT,),
        in_specs=[
            pl.BlockSpec((SROWS, D),
                         lambda t, v: (jnp.where(v[t] == 1, t, 0), 0)),
            pl.BlockSpec((1, D, H), lambda t, v: (t // CTILE, 0, 0)),
            pl.BlockSpec((1, 1, H), lambda t, v: (t // CTILE, 0, 0)),
            pl.BlockSpec((1, H, D), lambda t, v: (t // CTILE, 0, 0)),
            pl.BlockSpec((1, 1, D), lambda t, v: (t // CTILE, 0, 0)),
        ],
        out_specs=pl.BlockSpec((SROWS, D),
                               lambda t, v: (jnp.where(v[t] == 1, t, NT), 0)),
    )
    return pl.pallas_call(
        _moe_body,
        grid_spec=grid_spec,
        out_shape=jax.ShapeDtypeStruct(((NT + 1) * SROWS, D), jnp.float32),
    )(valid_flat, disp, e_w1, e_b1.reshape(E, 1, H), e_w2,
      e_b2.reshape(E, 1, D))


def _comb_body(y0_ref, y1_ref, g0_ref, g1_ref, out_ref, pooled_ref):
    gw0 = g0_ref[...][:, :, None]
    gw1 = g1_ref[...][:, :, None]
    o = gw0 * y0_ref[...] + gw1 * y1_ref[...]
    out_ref[...] = o
    pooled_ref[...] = (o[:, 0, :] + o[:, 1, :] + o[:, 2, :]) * (1.0 / M)


def _combine(y0, y1, g0, g1):
    outs = (
        jax.ShapeDtypeStruct((B, M, D), jnp.float32),
        jax.ShapeDtypeStruct((B, D), jnp.float32),
    )
    return pl.pallas_call(
        _comb_body,
        grid=(NBT,),
        in_specs=[
            pl.BlockSpec((TB, M, D), lambda bt: (bt, 0, 0)),
            pl.BlockSpec((TB, M, D), lambda bt: (bt, 0, 0)),
            pl.BlockSpec((TB, 1), lambda bt: (bt, 0)),
            pl.BlockSpec((TB, 1), lambda bt: (bt, 0)),
        ],
        out_specs=(
            pl.BlockSpec((TB, M, D), lambda bt: (bt, 0, 0)),
            pl.BlockSpec((TB, D), lambda bt: (bt, 0)),
        ),
        out_shape=outs,
    )(y0, y1, g0, g1)


def kernel(mod0, mod1, mod2, p0_w, p0_b, p2_w, p2_b, ln_g, ln_b, gate_w,
           gate_b, e_w1, e_b1, e_w2, e_b2):
    (projected, tokens, gates, slot0, slot1, g0, g1, valid,
     bloss) = _k1(mod0, mod1, mod2, p0_w, p0_b, p2_w, p2_b, ln_g, ln_b,
                  gate_w, gate_b)
    tokens_flat = tokens.reshape(B * M, D)
    s0 = slot0.reshape(B)
    s1 = slot1.reshape(B)
    disp = _sc_scatter(tokens_flat, s0, s1)
    y = _sparse_moe(valid.reshape(NT), disp, e_w1, e_b1, e_w2, e_b2)
    y0, y1 = _sc_gather(y, s0, s1)
    modality_tokens, pooled = _combine(y0.reshape(B, M, D),
                                       y1.reshape(B, M, D), g0, g1)
    return pooled, modality_tokens, projected, gates, bloss[0, 0]


# speedup vs baseline: 2.6104x; 1.1104x over previous
import jax
import jax.numpy as jnp
from jax import lax
from jax.experimental import pallas as pl
from jax.experimental.pallas import tpu as pltpu
from jax.experimental.pallas import tpu_sc as plsc

B = 1024
D = 1024
M = 3
E = 8
H = 2048
TB = 128
NBT = B // TB
NTMAX = (2 * B) // TB + E
SROWS = TB * M
NDROWS = NTMAX * SROWS

NC = 2
NS = 16
NW = NC * NS
SPW = B // NW
WROWS = SPW * M


def _erf(x):
    a1, a2, a3, a4, a5 = (
        0.254829592, -0.284496736, 1.421413741, -1.453152027, 1.061405429)
    p = 0.3275911
    s = jnp.sign(x)
    ax = jnp.abs(x)
    t = 1.0 / (1.0 + p * ax)
    poly = t * (a1 + t * (a2 + t * (a3 + t * (a4 + t * a5))))
    y = 1.0 - poly * jnp.exp(-ax * ax)
    return s * y


def _gelu(x):
    return 0.5 * x * (1.0 + _erf(x * 0.7071067811865476))


def _k1_body(mod0_ref, mod1_ref, mod2_ref, p0w_ref, p0b_ref, p2w_ref,
             p2b_ref, lng_ref, lnb_ref, gw_ref, gb_ref,
             proj_ref, tok_ref, gates_ref, e0_ref, e1_ref, pos0_ref,
             pos1_ref, g0_ref, g1_ref, base_ref, texp_ref, nta_ref,
             bloss_ref, run_ref, imp_ref):
    bt = pl.program_id(0)
    bf = jnp.bfloat16

    def bdot(a, b):
        return lax.dot_general(a.astype(bf), b.astype(bf),
                               (((1,), (1,)), ((), ())),
                               preferred_element_type=jnp.float32)

    t0 = bdot(mod0_ref[...], p0w_ref[...]) + p0b_ref[...]
    t1 = mod1_ref[...]
    t2 = bdot(mod2_ref[...], p2w_ref[...]) + p2b_ref[...]

    proj_ref[:, 0, :] = t0
    proj_ref[:, 1, :] = t1
    proj_ref[:, 2, :] = t2

    def ln(t):
        mu = jnp.mean(t, axis=-1, keepdims=True)
        var = jnp.mean((t - mu) ** 2, axis=-1, keepdims=True)
        return (t - mu) / jnp.sqrt(var + 1e-5) * lng_ref[...] + lnb_ref[...]

    n0, n1, n2 = ln(t0), ln(t1), ln(t2)
    tok_ref[:, 0, :] = n0
    tok_ref[:, 1, :] = n1
    tok_ref[:, 2, :] = n2

    ctx = jnp.concatenate([n0, n1, n2], axis=-1)
    logits = bdot(ctx, gw_ref[...]) + gb_ref[...]

    eidx = lax.broadcasted_iota(jnp.int32, (TB, E), 1)
    m1 = jnp.max(logits, axis=-1, keepdims=True)
    idx1 = jnp.min(jnp.where(logits == m1, eidx, E), axis=-1, keepdims=True)
    oh1 = eidx == idx1
    masked = jnp.where(oh1, -jnp.inf, logits)
    m2 = jnp.max(masked, axis=-1, keepdims=True)
    idx2 = jnp.min(jnp.where(masked == m2, eidx, E), axis=-1, keepdims=True)
    oh2 = eidx == idx2

    ex = jnp.exp(m2 - m1)
    den = 1.0 + ex
    g1v = 1.0 / den
    g2v = ex / den
    gates = jnp.where(oh1, g1v, jnp.where(oh2, g2v, 0.0))
    gates_ref[...] = gates

    @pl.when(bt == 0)
    def _():
        run_ref[...] = jnp.zeros_like(run_ref)
        imp_ref[...] = jnp.zeros_like(imp_ref)

    mask = (oh1 | oh2).astype(jnp.float32)
    bi = lax.broadcasted_iota(jnp.int32, (TB, TB), 0)
    bj = lax.broadcasted_iota(jnp.int32, (TB, TB), 1)
    tri = (bj < bi).astype(jnp.bfloat16)
    pos = lax.dot_general(tri, mask.astype(jnp.bfloat16),
                          (((1,), (0,)), ((), ())),
                          preferred_element_type=jnp.float32)
    posg = pos + run_ref[...]

    pos1v = jnp.sum(jnp.where(oh1, posg, 0.0), axis=-1).astype(jnp.int32)
    pos2v = jnp.sum(jnp.where(oh2, posg, 0.0), axis=-1).astype(jnp.int32)
    e0_ref[...] = idx1
    e1_ref[...] = idx2
    pos0_ref[...] = pos1v[:, None]
    pos1_ref[...] = pos2v[:, None]
    g0_ref[...] = g1v
    g1_ref[...] = g2v

    run_ref[...] = run_ref[...] + jnp.sum(mask, axis=0, keepdims=True)
    imp_ref[...] = imp_ref[...] + jnp.sum(gates, axis=0, keepdims=True)

    @pl.when(bt == NBT - 1)
    def _():
        cntf = run_ref[...]
        ntiles = jnp.floor((cntf + (TB - 1)) * (1.0 / TB))
        tri8a = lax.broadcasted_iota(jnp.int32, (E, E), 0)
        tri8b = lax.broadcasted_iota(jnp.int32, (E, E), 1)
        incl = (tri8a <= tri8b).astype(jnp.bfloat16)
        ends = lax.dot_general(ntiles.astype(jnp.bfloat16), incl,
                               (((1,), (0,)), ((), ())),
                               preferred_element_type=jnp.float32)
        ends_i = ends.astype(jnp.int32)
        ntiles_i = ntiles.astype(jnp.int32)
        base_t = ends_i - ntiles_i
        brow = base_t * SROWS
        base_ref[...] = jnp.concatenate(
            [brow, jnp.zeros((1, E), jnp.int32)], axis=1)
        ti = lax.broadcasted_iota(jnp.int32, (NTMAX, E), 0)
        texp = jnp.sum((jnp.broadcast_to(ends_i, (NTMAX, E)) <= ti)
                       .astype(jnp.int32), axis=1, keepdims=True)
        texp_ref[...] = jnp.minimum(texp, E - 1)
        nta_ref[...] = ends_i[:, E - 1:E]
        imp = imp_ref[...]
        mu = jnp.mean(imp)
        var = jnp.mean((imp - mu) ** 2)
        bloss_ref[...] = jnp.reshape(0.01 * var / (mu * mu + 1e-10), (1, 1))


def _k1(mod0, mod1, mod2, p0_w, p0_b, p2_w, p2_b, ln_g, ln_b, gate_w, gate_b):
    full = lambda s: pl.BlockSpec(s, lambda bt: (0,) * len(s))
    outs = (
        jax.ShapeDtypeStruct((B, M, D), jnp.float32),
        jax.ShapeDtypeStruct((B, M, D), jnp.float32),
        jax.ShapeDtypeStruct((B, E), jnp.float32),
        jax.ShapeDtypeStruct((B, 1), jnp.int32),
        jax.ShapeDtypeStruct((B, 1), jnp.int32),
        jax.ShapeDtypeStruct((B, 1), jnp.int32),
        jax.ShapeDtypeStruct((B, 1), jnp.int32),
        jax.ShapeDtypeStruct((B, 1), jnp.float32),
        jax.ShapeDtypeStruct((B, 1), jnp.float32),
        jax.ShapeDtypeStruct((1, 2 * E), jnp.int32),
        jax.ShapeDtypeStruct((NTMAX, 1), jnp.int32),
        jax.ShapeDtypeStruct((1, 1), jnp.int32),
        jax.ShapeDtypeStruct((1, 1), jnp.float32),
    )
    return pl.pallas_call(
        _k1_body,
        grid=(NBT,),
        in_specs=[
            pl.BlockSpec((TB, 768), lambda bt: (bt, 0)),
            pl.BlockSpec((TB, D), lambda bt: (bt, 0)),
            pl.BlockSpec((TB, 512), lambda bt: (bt, 0)),
            full((D, 768)),
            full((1, D)),
            full((D, 512)),
            full((1, D)),
            full((1, D)),
            full((1, D)),
            full((E, M * D)),
            full((1, E)),
        ],
        out_specs=(
            pl.BlockSpec((TB, M, D), lambda bt: (bt, 0, 0)),
            pl.BlockSpec((TB, M, D), lambda bt: (bt, 0, 0)),
            pl.BlockSpec((TB, E), lambda bt: (bt, 0)),
            pl.BlockSpec((TB, 1), lambda bt: (bt, 0)),
            pl.BlockSpec((TB, 1), lambda bt: (bt, 0)),
            pl.BlockSpec((TB, 1), lambda bt: (bt, 0)),
            pl.BlockSpec((TB, 1), lambda bt: (bt, 0)),
            pl.BlockSpec((TB, 1), lambda bt: (bt, 0)),
            pl.BlockSpec((TB, 1), lambda bt: (bt, 0)),
            full((1, 2 * E)),
            full((NTMAX, 1)),
            full((1, 1)),
            full((1, 1)),
        ),
        out_shape=outs,
        scratch_shapes=[
            pltpu.VMEM((1, E), jnp.float32),
            pltpu.VMEM((1, E), jnp.float32),
        ],
    )(mod0, mod1, mod2, p0_w, p0_b.reshape(1, D), p2_w, p2_b.reshape(1, D),
      ln_g.reshape(1, D), ln_b.reshape(1, D), gate_w, gate_b.reshape(1, E))


def _build_idx(idx_ref, se_ref, sp_ref, bvm_ref):
    it = lax.broadcasted_iota(jnp.int32, (16,), 0)
    for g in range(SPW // 16):
        ev = se_ref[pl.ds(g * 16, 16)]
        pv = sp_ref[pl.ds(g * 16, 16)]
        bv = plsc.load_gather(bvm_ref, [ev])
        rowbase = bv + pv * 3
        for r in range(M):
            plsc.store_scatter(idx_ref, [it * 3 + (g * 48 + r)], rowbase + r)


def _sc_scatter_body(tok_hbm, e0_hbm, e1_hbm, p0_hbm, p1_hbm, base_hbm,
                     disp_hbm, buf, idx, se, sp, bvm, sem):
    wid = lax.axis_index("s") * NC + lax.axis_index("c")
    base = wid * SPW
    pltpu.sync_copy(tok_hbm.at[pl.ds(base * M, WROWS)], buf)
    pltpu.sync_copy(base_hbm, bvm)
    pltpu.sync_copy(e0_hbm.at[pl.ds(base, SPW)], se)
    pltpu.sync_copy(p0_hbm.at[pl.ds(base, SPW)], sp)
    _build_idx(idx, se, sp, bvm)
    pltpu.async_copy(buf, disp_hbm.at[idx], sem).wait()
    pltpu.sync_copy(e1_hbm.at[pl.ds(base, SPW)], se)
    pltpu.sync_copy(p1_hbm.at[pl.ds(base, SPW)], sp)
    _build_idx(idx, se, sp, bvm)
    pltpu.async_copy(buf, disp_hbm.at[idx], sem).wait()


def _sc_scatter(tokens_flat, e0, e1, p0, p1, base_row):
    mesh = plsc.VectorSubcoreMesh(core_axis_name="c", subcore_axis_name="s")
    return pl.kernel(
        _sc_scatter_body,
        out_type=jax.ShapeDtypeStruct((NDROWS, D), jnp.float32),
        mesh=mesh,
        compiler_params=pltpu.CompilerParams(needs_layout_passes=False),
        scratch_types=[
            pltpu.VMEM((WROWS, D), jnp.float32),
            pltpu.VMEM((WROWS,), jnp.int32),
            pltpu.VMEM((SPW,), jnp.int32),
            pltpu.VMEM((SPW,), jnp.int32),
            pltpu.VMEM((2 * E,), jnp.int32),
            pltpu.SemaphoreType.DMA,
        ],
    )(tokens_flat, e0, e1, p0, p1, base_row)


def _sc_gather_body(y_hbm, e0_hbm, e1_hbm, p0_hbm, p1_hbm, base_hbm,
                    y0_hbm, y1_hbm, buf, idx, se, sp, bvm, sem):
    wid = lax.axis_index("s") * NC + lax.axis_index("c")
    base = wid * SPW
    pltpu.sync_copy(base_hbm, bvm)
    pltpu.sync_copy(e0_hbm.at[pl.ds(base, SPW)], se)
    pltpu.sync_copy(p0_hbm.at[pl.ds(base, SPW)], sp)
    _build_idx(idx, se, sp, bvm)
    pltpu.async_copy(y_hbm.at[idx], buf, sem).wait()
    pltpu.sync_copy(buf, y0_hbm.at[pl.ds(base * M, WROWS)])
    pltpu.sync_copy(e1_hbm.at[pl.ds(base, SPW)], se)
    pltpu.sync_copy(p1_hbm.at[pl.ds(base, SPW)], sp)
    _build_idx(idx, se, sp, bvm)
    pltpu.async_copy(y_hbm.at[idx], buf, sem).wait()
    pltpu.sync_copy(buf, y1_hbm.at[pl.ds(base * M, WROWS)])


def _sc_gather(y, e0, e1, p0, p1, base_row):
    mesh = plsc.VectorSubcoreMesh(core_axis_name="c", subcore_axis_name="s")
    return pl.kernel(
        _sc_gather_body,
        out_type=(jax.ShapeDtypeStruct((B * M, D), jnp.float32),
                  jax.ShapeDtypeStruct((B * M, D), jnp.float32)),
        mesh=mesh,
        compiler_params=pltpu.CompilerParams(needs_layout_passes=False),
        scratch_types=[
            pltpu.VMEM((WROWS, D), jnp.float32),
            pltpu.VMEM((WROWS,), jnp.int32),
            pltpu.VMEM((SPW,), jnp.int32),
            pltpu.VMEM((SPW,), jnp.int32),
            pltpu.VMEM((2 * E,), jnp.int32),
            pltpu.SemaphoreType.DMA,
        ],
    )(y, e0, e1, p0, p1, base_row)


def _moe_body(texp_ref, nta_ref, x_ref, w1_ref, b1_ref, w2_ref, b2_ref,
              y_ref):
    t = pl.program_id(0)

    @pl.when(t < nta_ref[0])
    def _():
        x = x_ref[...].astype(jnp.bfloat16)
        w1 = w1_ref[0].astype(jnp.bfloat16)
        h = jnp.dot(x, w1, preferred_element_type=jnp.float32) + b1_ref[0]
        h = _gelu(h)
        w2 = w2_ref[0].astype(jnp.bfloat16)
        y_ref[...] = (jnp.dot(h.astype(jnp.bfloat16), w2,
                              preferred_element_type=jnp.float32) + b2_ref[0])


def _sparse_moe(texp, nta, disp, e_w1, e_b1, e_w2, e_b2):
    grid_spec = pltpu.PrefetchScalarGridSpec(
        num_scalar_prefetch=2,
        grid=(NTMAX,),
        in_specs=[
            pl.BlockSpec((SROWS, D), lambda t, te, na: (t, 0)),
            pl.BlockSpec((1, D, H), lambda t, te, na: (te[t], 0, 0)),
            pl.BlockSpec((1, 1, H), lambda t, te, na: (te[t], 0, 0)),
            pl.BlockSpec((1, H, D), lambda t, te, na: (te[t], 0, 0)),
            pl.BlockSpec((1, 1, D), lambda t, te, na: (te[t], 0, 0)),
        ],
        out_specs=pl.BlockSpec(
            (SROWS, D), lambda t, te, na: (jnp.where(t < na[0], t, NTMAX), 0)),
    )
    return pl.pallas_call(
        _moe_body,
        grid_spec=grid_spec,
        out_shape=jax.ShapeDtypeStruct(((NTMAX + 1) * SROWS, D), jnp.float32),
    )(texp, nta, disp, e_w1, e_b1.reshape(E, 1, H), e_w2,
      e_b2.reshape(E, 1, D))


def _comb_body(y0_ref, y1_ref, g0_ref, g1_ref, out_ref, pooled_ref):
    gw0 = g0_ref[...][:, :, None]
    gw1 = g1_ref[...][:, :, None]
    o = gw0 * y0_ref[...] + gw1 * y1_ref[...]
    out_ref[...] = o
    pooled_ref[...] = (o[:, 0, :] + o[:, 1, :] + o[:, 2, :]) * (1.0 / M)


def _combine(y0, y1, g0, g1):
    outs = (
        jax.ShapeDtypeStruct((B, M, D), jnp.float32),
        jax.ShapeDtypeStruct((B, D), jnp.float32),
    )
    return pl.pallas_call(
        _comb_body,
        grid=(NBT,),
        in_specs=[
            pl.BlockSpec((TB, M, D), lambda bt: (bt, 0, 0)),
            pl.BlockSpec((TB, M, D), lambda bt: (bt, 0, 0)),
            pl.BlockSpec((TB, 1), lambda bt: (bt, 0)),
            pl.BlockSpec((TB, 1), lambda bt: (bt, 0)),
        ],
        out_specs=(
            pl.BlockSpec((TB, M, D), lambda bt: (bt, 0, 0)),
            pl.BlockSpec((TB, D), lambda bt: (bt, 0)),
        ),
        out_shape=outs,
    )(y0, y1, g0, g1)


def kernel(mod0, mod1, mod2, p0_w, p0_b, p2_w, p2_b, ln_g, ln_b, gate_w,
           gate_b, e_w1, e_b1, e_w2, e_b2):
    (projected, tokens, gates, e0, e1, pos0, pos1, g0, g1, base_row, texp,
     nta, bloss) = _k1(mod0, mod1, mod2, p0_w, p0_b, p2_w, p2_b, ln_g, ln_b,
                       gate_w, gate_b)
    tokens_flat = tokens.reshape(B * M, D)
    e0f = e0.reshape(B)
    e1f = e1.reshape(B)
    p0f = pos0.reshape(B)
    p1f = pos1.reshape(B)
    basef = base_row.reshape(2 * E)
    disp = _sc_scatter(tokens_flat, e0f, e1f, p0f, p1f, basef)
    y = _sparse_moe(texp.reshape(NTMAX), nta.reshape(1), disp,
                    e_w1, e_b1, e_w2, e_b2)
    y0, y1 = _sc_gather(y, e0f, e1f, p0f, p1f, basef)
    modality_tokens, pooled = _combine(y0.reshape(B, M, D),
                                       y1.reshape(B, M, D), g0, g1)
    return pooled, modality_tokens, projected, gates, bloss[0, 0]


# speedup vs baseline: 2.7808x; 1.0653x over previous
import jax
import jax.numpy as jnp
from jax import lax
from jax.experimental import pallas as pl
from jax.experimental.pallas import tpu as pltpu
from jax.experimental.pallas import tpu_sc as plsc

B = 1024
D = 1024
M = 3
E = 8
H = 2048
TB = 128
NBT = B // TB
NTMAX = (2 * B) // TB + E
SROWS = TB * M
NDROWS = NTMAX * SROWS

NC = 2
NS = 16
NW = NC * NS
SPW = B // NW
WROWS = SPW * M


def _erf(x):
    a1, a2, a3, a4, a5 = (
        0.254829592, -0.284496736, 1.421413741, -1.453152027, 1.061405429)
    p = 0.3275911
    s = jnp.sign(x)
    ax = jnp.abs(x)
    t = 1.0 / (1.0 + p * ax)
    poly = t * (a1 + t * (a2 + t * (a3 + t * (a4 + t * a5))))
    y = 1.0 - poly * jnp.exp(-ax * ax)
    return s * y


def _gelu(x):
    return 0.5 * x * (1.0 + _erf(x * 0.7071067811865476))


def _k1_body(mod0_ref, mod1_ref, mod2_ref, p0w_ref, p0b_ref, p2w_ref,
             p2b_ref, lng_ref, lnb_ref, gw_ref, gb_ref,
             proj_ref, tok_ref, gates_ref, e0_ref, e1_ref, pos0_ref,
             pos1_ref, g0_ref, g1_ref, base_ref, texp_ref, nta_ref,
             bloss_ref, run_ref, imp_ref):
    bt = pl.program_id(0)
    bf = jnp.bfloat16

    def bdot(a, b):
        return lax.dot_general(a.astype(bf), b.astype(bf),
                               (((1,), (1,)), ((), ())),
                               preferred_element_type=jnp.float32)

    t0 = bdot(mod0_ref[...], p0w_ref[...]) + p0b_ref[...]
    t1 = mod1_ref[...]
    t2 = bdot(mod2_ref[...], p2w_ref[...]) + p2b_ref[...]

    proj_ref[:, 0, :] = t0
    proj_ref[:, 1, :] = t1
    proj_ref[:, 2, :] = t2

    def ln(t):
        mu = jnp.mean(t, axis=-1, keepdims=True)
        var = jnp.mean((t - mu) ** 2, axis=-1, keepdims=True)
        return (t - mu) / jnp.sqrt(var + 1e-5) * lng_ref[...] + lnb_ref[...]

    n0, n1, n2 = ln(t0), ln(t1), ln(t2)
    tok_ref[:, 0, :] = n0
    tok_ref[:, 1, :] = n1
    tok_ref[:, 2, :] = n2

    ctx = jnp.concatenate([n0, n1, n2], axis=-1)
    logits = bdot(ctx, gw_ref[...]) + gb_ref[...]

    eidx = lax.broadcasted_iota(jnp.int32, (TB, E), 1)
    m1 = jnp.max(logits, axis=-1, keepdims=True)
    idx1 = jnp.min(jnp.where(logits == m1, eidx, E), axis=-1, keepdims=True)
    oh1 = eidx == idx1
    masked = jnp.where(oh1, -jnp.inf, logits)
    m2 = jnp.max(masked, axis=-1, keepdims=True)
    idx2 = jnp.min(jnp.where(masked == m2, eidx, E), axis=-1, keepdims=True)
    oh2 = eidx == idx2

    ex = jnp.exp(m2 - m1)
    den = 1.0 + ex
    g1v = 1.0 / den
    g2v = ex / den
    gates = jnp.where(oh1, g1v, jnp.where(oh2, g2v, 0.0))
    gates_ref[...] = gates

    @pl.when(bt == 0)
    def _():
        run_ref[...] = jnp.zeros_like(run_ref)
        imp_ref[...] = jnp.zeros_like(imp_ref)

    mask = (oh1 | oh2).astype(jnp.float32)
    bi = lax.broadcasted_iota(jnp.int32, (TB, TB), 0)
    bj = lax.broadcasted_iota(jnp.int32, (TB, TB), 1)
    tri = (bj < bi).astype(jnp.bfloat16)
    pos = lax.dot_general(tri, mask.astype(jnp.bfloat16),
                          (((1,), (0,)), ((), ())),
                          preferred_element_type=jnp.float32)
    posg = pos + run_ref[...]

    pos1v = jnp.sum(jnp.where(oh1, posg, 0.0), axis=-1).astype(jnp.int32)
    pos2v = jnp.sum(jnp.where(oh2, posg, 0.0), axis=-1).astype(jnp.int32)
    e0_ref[...] = idx1
    e1_ref[...] = idx2
    pos0_ref[...] = pos1v[:, None]
    pos1_ref[...] = pos2v[:, None]
    g0_ref[...] = g1v
    g1_ref[...] = g2v

    run_ref[...] = run_ref[...] + jnp.sum(mask, axis=0, keepdims=True)
    imp_ref[...] = imp_ref[...] + jnp.sum(gates, axis=0, keepdims=True)

    @pl.when(bt == NBT - 1)
    def _():
        cntf = run_ref[...]
        ntiles = jnp.floor((cntf + (TB - 1)) * (1.0 / TB))
        tri8a = lax.broadcasted_iota(jnp.int32, (E, E), 0)
        tri8b = lax.broadcasted_iota(jnp.int32, (E, E), 1)
        incl = (tri8a <= tri8b).astype(jnp.bfloat16)
        ends = lax.dot_general(ntiles.astype(jnp.bfloat16), incl,
                               (((1,), (0,)), ((), ())),
                               preferred_element_type=jnp.float32)
        ends_i = ends.astype(jnp.int32)
        ntiles_i = ntiles.astype(jnp.int32)
        base_t = ends_i - ntiles_i
        brow = base_t * SROWS
        base_ref[...] = jnp.concatenate(
            [brow, jnp.zeros((1, E), jnp.int32)], axis=1)
        ti = lax.broadcasted_iota(jnp.int32, (NTMAX, E), 0)
        texp = jnp.sum((jnp.broadcast_to(ends_i, (NTMAX, E)) <= ti)
                       .astype(jnp.int32), axis=1, keepdims=True)
        texp_ref[...] = jnp.minimum(texp, E - 1)
        nta_ref[...] = ends_i[:, E - 1:E]
        imp = imp_ref[...]
        mu = jnp.mean(imp)
        var = jnp.mean((imp - mu) ** 2)
        bloss_ref[...] = jnp.reshape(0.01 * var / (mu * mu + 1e-10), (1, 1))


def _k1(mod0, mod1, mod2, p0_w, p0_b, p2_w, p2_b, ln_g, ln_b, gate_w, gate_b):
    full = lambda s: pl.BlockSpec(s, lambda bt: (0,) * len(s))
    outs = (
        jax.ShapeDtypeStruct((B, M, D), jnp.float32),
        jax.ShapeDtypeStruct((B, M, D), jnp.float32),
        jax.ShapeDtypeStruct((B, E), jnp.float32),
        jax.ShapeDtypeStruct((B, 1), jnp.int32),
        jax.ShapeDtypeStruct((B, 1), jnp.int32),
        jax.ShapeDtypeStruct((B, 1), jnp.int32),
        jax.ShapeDtypeStruct((B, 1), jnp.int32),
        jax.ShapeDtypeStruct((B, 1), jnp.float32),
        jax.ShapeDtypeStruct((B, 1), jnp.float32),
        jax.ShapeDtypeStruct((1, 2 * E), jnp.int32),
        jax.ShapeDtypeStruct((NTMAX, 1), jnp.int32),
        jax.ShapeDtypeStruct((1, 1), jnp.int32),
        jax.ShapeDtypeStruct((1, 1), jnp.float32),
    )
    return pl.pallas_call(
        _k1_body,
        grid=(NBT,),
        in_specs=[
            pl.BlockSpec((TB, 768), lambda bt: (bt, 0)),
            pl.BlockSpec((TB, D), lambda bt: (bt, 0)),
            pl.BlockSpec((TB, 512), lambda bt: (bt, 0)),
            full((D, 768)),
            full((1, D)),
            full((D, 512)),
            full((1, D)),
            full((1, D)),
            full((1, D)),
            full((E, M * D)),
            full((1, E)),
        ],
        out_specs=(
            pl.BlockSpec((TB, M, D), lambda bt: (bt, 0, 0)),
            pl.BlockSpec((TB, M, D), lambda bt: (bt, 0, 0)),
            pl.BlockSpec((TB, E), lambda bt: (bt, 0)),
            pl.BlockSpec((TB, 1), lambda bt: (bt, 0)),
            pl.BlockSpec((TB, 1), lambda bt: (bt, 0)),
            pl.BlockSpec((TB, 1), lambda bt: (bt, 0)),
            pl.BlockSpec((TB, 1), lambda bt: (bt, 0)),
            pl.BlockSpec((TB, 1), lambda bt: (bt, 0)),
            pl.BlockSpec((TB, 1), lambda bt: (bt, 0)),
            full((1, 2 * E)),
            full((NTMAX, 1)),
            full((1, 1)),
            full((1, 1)),
        ),
        out_shape=outs,
        scratch_shapes=[
            pltpu.VMEM((1, E), jnp.float32),
            pltpu.VMEM((1, E), jnp.float32),
        ],
    )(mod0, mod1, mod2, p0_w, p0_b.reshape(1, D), p2_w, p2_b.reshape(1, D),
      ln_g.reshape(1, D), ln_b.reshape(1, D), gate_w, gate_b.reshape(1, E))


def _build_idx(idx_ref, se_ref, sp_ref, bvm_ref):
    it = lax.broadcasted_iota(jnp.int32, (16,), 0)
    for g in range(SPW // 16):
        ev = se_ref[pl.ds(g * 16, 16)]
        pv = sp_ref[pl.ds(g * 16, 16)]
        bv = plsc.load_gather(bvm_ref, [ev])
        rowbase = bv + pv * 3
        for r in range(M):
            plsc.store_scatter(idx_ref, [it * 3 + (g * 48 + r)], rowbase + r)


def _sc_scatter_body(tok_hbm, e0_hbm, e1_hbm, p0_hbm, p1_hbm, base_hbm,
                     disp_hbm, buf, idx, se, sp, bvm, sem):
    wid = lax.axis_index("s") * NC + lax.axis_index("c")
    base = wid * SPW
    pltpu.sync_copy(tok_hbm.at[pl.ds(base * M, WROWS)], buf)
    pltpu.sync_copy(base_hbm, bvm)
    pltpu.sync_copy(e0_hbm.at[pl.ds(base, SPW)], se)
    pltpu.sync_copy(p0_hbm.at[pl.ds(base, SPW)], sp)
    _build_idx(idx, se, sp, bvm)
    pltpu.async_copy(buf, disp_hbm.at[idx], sem).wait()
    pltpu.sync_copy(e1_hbm.at[pl.ds(base, SPW)], se)
    pltpu.sync_copy(p1_hbm.at[pl.ds(base, SPW)], sp)
    _build_idx(idx, se, sp, bvm)
    pltpu.async_copy(buf, disp_hbm.at[idx], sem).wait()


def _sc_scatter(tokens_flat, e0, e1, p0, p1, base_row):
    mesh = plsc.VectorSubcoreMesh(core_axis_name="c", subcore_axis_name="s")
    return pl.kernel(
        _sc_scatter_body,
        out_type=jax.ShapeDtypeStruct((NDROWS, D), jnp.float32),
        mesh=mesh,
        compiler_params=pltpu.CompilerParams(needs_layout_passes=False),
        scratch_types=[
            pltpu.VMEM((WROWS, D), jnp.float32),
            pltpu.VMEM((WROWS,), jnp.int32),
            pltpu.VMEM((SPW,), jnp.int32),
            pltpu.VMEM((SPW,), jnp.int32),
            pltpu.VMEM((2 * E,), jnp.int32),
            pltpu.SemaphoreType.DMA,
        ],
    )(tokens_flat, e0, e1, p0, p1, base_row)


def _sc_gather_body(y_hbm, e0_hbm, e1_hbm, p0_hbm, p1_hbm, base_hbm,
                    y0_hbm, y1_hbm, buf, idx, se, sp, bvm, sem):
    wid = lax.axis_index("s") * NC + lax.axis_index("c")
    base = wid * SPW
    pltpu.sync_copy(base_hbm, bvm)
    pltpu.sync_copy(e0_hbm.at[pl.ds(base, SPW)], se)
    pltpu.sync_copy(p0_hbm.at[pl.ds(base, SPW)], sp)
    _build_idx(idx, se, sp, bvm)
    pltpu.async_copy(y_hbm.at[idx], buf, sem).wait()
    pltpu.sync_copy(buf, y0_hbm.at[pl.ds(base * M, WROWS)])
    pltpu.sync_copy(e1_hbm.at[pl.ds(base, SPW)], se)
    pltpu.sync_copy(p1_hbm.at[pl.ds(base, SPW)], sp)
    _build_idx(idx, se, sp, bvm)
    pltpu.async_copy(y_hbm.at[idx], buf, sem).wait()
    pltpu.sync_copy(buf, y1_hbm.at[pl.ds(base * M, WROWS)])


def _sc_gather(y, e0, e1, p0, p1, base_row):
    mesh = plsc.VectorSubcoreMesh(core_axis_name="c", subcore_axis_name="s")
    return pl.kernel(
        _sc_gather_body,
        out_type=(jax.ShapeDtypeStruct((B * M, D), jnp.float32),
                  jax.ShapeDtypeStruct((B * M, D), jnp.float32)),
        mesh=mesh,
        compiler_params=pltpu.CompilerParams(needs_layout_passes=False),
        scratch_types=[
            pltpu.VMEM((WROWS, D), jnp.float32),
            pltpu.VMEM((WROWS,), jnp.int32),
            pltpu.VMEM((SPW,), jnp.int32),
            pltpu.VMEM((SPW,), jnp.int32),
            pltpu.VMEM((2 * E,), jnp.int32),
            pltpu.SemaphoreType.DMA,
        ],
    )(y, e0, e1, p0, p1, base_row)


def _moe_body(texp_ref, nta_ref, x_ref, w1_ref, b1_ref, w2_ref, b2_ref,
              y_ref):
    t = pl.program_id(0)

    @pl.when(t < nta_ref[0])
    def _():
        x = x_ref[...].astype(jnp.bfloat16)
        w1 = w1_ref[0].astype(jnp.bfloat16)
        h = jnp.dot(x, w1, preferred_element_type=jnp.float32) + b1_ref[0]
        hb = h.astype(jnp.bfloat16)
        u = hb * jnp.bfloat16(0.7978845608) * (
            jnp.bfloat16(1.0) + jnp.bfloat16(0.044715) * hb * hb)
        g = jnp.bfloat16(0.5) * hb * (jnp.bfloat16(1.0) + jnp.tanh(u))
        w2 = w2_ref[0].astype(jnp.bfloat16)
        y_ref[...] = (jnp.dot(g, w2,
                              preferred_element_type=jnp.float32) + b2_ref[0])


def _sparse_moe(texp, nta, disp, e_w1, e_b1, e_w2, e_b2):
    grid_spec = pltpu.PrefetchScalarGridSpec(
        num_scalar_prefetch=2,
        grid=(NTMAX,),
        in_specs=[
            pl.BlockSpec((SROWS, D), lambda t, te, na: (t, 0)),
            pl.BlockSpec((1, D, H), lambda t, te, na: (te[t], 0, 0)),
            pl.BlockSpec((1, 1, H), lambda t, te, na: (te[t], 0, 0)),
            pl.BlockSpec((1, H, D), lambda t, te, na: (te[t], 0, 0)),
            pl.BlockSpec((1, 1, D), lambda t, te, na: (te[t], 0, 0)),
        ],
        out_specs=pl.BlockSpec(
            (SROWS, D), lambda t, te, na: (jnp.where(t < na[0], t, NTMAX), 0)),
    )
    return pl.pallas_call(
        _moe_body,
        grid_spec=grid_spec,
        out_shape=jax.ShapeDtypeStruct(((NTMAX + 1) * SROWS, D), jnp.float32),
    )(texp, nta, disp, e_w1, e_b1.reshape(E, 1, H), e_w2,
      e_b2.reshape(E, 1, D))


def _comb_body(y0_ref, y1_ref, g0_ref, g1_ref, out_ref, pooled_ref):
    gw0 = g0_ref[...][:, :, None]
    gw1 = g1_ref[...][:, :, None]
    o = gw0 * y0_ref[...] + gw1 * y1_ref[...]
    out_ref[...] = o
    pooled_ref[...] = (o[:, 0, :] + o[:, 1, :] + o[:, 2, :]) * (1.0 / M)


def _combine(y0, y1, g0, g1):
    outs = (
        jax.ShapeDtypeStruct((B, M, D), jnp.float32),
        jax.ShapeDtypeStruct((B, D), jnp.float32),
    )
    return pl.pallas_call(
        _comb_body,
        grid=(NBT,),
        in_specs=[
            pl.BlockSpec((TB, M, D), lambda bt: (bt, 0, 0)),
            pl.BlockSpec((TB, M, D), lambda bt: (bt, 0, 0)),
            pl.BlockSpec((TB, 1), lambda bt: (bt, 0)),
            pl.BlockSpec((TB, 1), lambda bt: (bt, 0)),
        ],
        out_specs=(
            pl.BlockSpec((TB, M, D), lambda bt: (bt, 0, 0)),
            pl.BlockSpec((TB, D), lambda bt: (bt, 0)),
        ),
        out_shape=outs,
    )(y0, y1, g0, g1)


def kernel(mod0, mod1, mod2, p0_w, p0_b, p2_w, p2_b, ln_g, ln_b, gate_w,
           gate_b, e_w1, e_b1, e_w2, e_b2):
    (projected, tokens, gates, e0, e1, pos0, pos1, g0, g1, base_row, texp,
     nta, bloss) = _k1(mod0, mod1, mod2, p0_w, p0_b, p2_w, p2_b, ln_g, ln_b,
                       gate_w, gate_b)
    tokens_flat = tokens.reshape(B * M, D)
    e0f = e0.reshape(B)
    e1f = e1.reshape(B)
    p0f = pos0.reshape(B)
    p1f = pos1.reshape(B)
    basef = base_row.reshape(2 * E)
    disp = _sc_scatter(tokens_flat, e0f, e1f, p0f, p1f, basef)
    y = _sparse_moe(texp.reshape(NTMAX), nta.reshape(1), disp,
                    e_w1, e_b1, e_w2, e_b2)
    y0, y1 = _sc_gather(y, e0f, e1f, p0f, p1f, basef)
    modality_tokens, pooled = _combine(y0.reshape(B, M, D),
                                       y1.reshape(B, M, D), g0, g1)
    return pooled, modality_tokens, projected, gates, bloss[0, 0]


# speedup vs baseline: 2.9171x; 1.0490x over previous
import jax
import jax.numpy as jnp
from jax import lax
from jax.experimental import pallas as pl
from jax.experimental.pallas import tpu as pltpu
from jax.experimental.pallas import tpu_sc as plsc

B = 1024
D = 1024
M = 3
E = 8
H = 2048
TB = 128
NBT = B // TB
TMOE = 256
NTMAX = (2 * B) // TMOE + E
SROWS = TMOE * M
NDROWS = NTMAX * SROWS

NC = 2
NS = 16
NW = NC * NS
SPW = B // NW
WROWS = SPW * M


def _erf(x):
    a1, a2, a3, a4, a5 = (
        0.254829592, -0.284496736, 1.421413741, -1.453152027, 1.061405429)
    p = 0.3275911
    s = jnp.sign(x)
    ax = jnp.abs(x)
    t = 1.0 / (1.0 + p * ax)
    poly = t * (a1 + t * (a2 + t * (a3 + t * (a4 + t * a5))))
    y = 1.0 - poly * jnp.exp(-ax * ax)
    return s * y


def _gelu(x):
    return 0.5 * x * (1.0 + _erf(x * 0.7071067811865476))


def _k1_body(mod0_ref, mod1_ref, mod2_ref, p0w_ref, p0b_ref, p2w_ref,
             p2b_ref, lng_ref, lnb_ref, gw_ref, gb_ref,
             proj_ref, tok_ref, gates_ref, e0_ref, e1_ref, pos0_ref,
             pos1_ref, g0_ref, g1_ref, base_ref, texp_ref, nta_ref,
             bloss_ref, run_ref, imp_ref):
    bt = pl.program_id(0)
    bf = jnp.bfloat16

    def bdot(a, b):
        return lax.dot_general(a.astype(bf), b.astype(bf),
                               (((1,), (1,)), ((), ())),
                               preferred_element_type=jnp.float32)

    t0 = bdot(mod0_ref[...], p0w_ref[...]) + p0b_ref[...]
    t1 = mod1_ref[...]
    t2 = bdot(mod2_ref[...], p2w_ref[...]) + p2b_ref[...]

    proj_ref[:, 0, :] = t0
    proj_ref[:, 1, :] = t1
    proj_ref[:, 2, :] = t2

    def ln(t):
        mu = jnp.mean(t, axis=-1, keepdims=True)
        var = jnp.mean((t - mu) ** 2, axis=-1, keepdims=True)
        return (t - mu) / jnp.sqrt(var + 1e-5) * lng_ref[...] + lnb_ref[...]

    n0, n1, n2 = ln(t0), ln(t1), ln(t2)
    tok_ref[:, 0, :] = n0
    tok_ref[:, 1, :] = n1
    tok_ref[:, 2, :] = n2

    ctx = jnp.concatenate([n0, n1, n2], axis=-1)
    logits = bdot(ctx, gw_ref[...]) + gb_ref[...]

    eidx = lax.broadcasted_iota(jnp.int32, (TB, E), 1)
    m1 = jnp.max(logits, axis=-1, keepdims=True)
    idx1 = jnp.min(jnp.where(logits == m1, eidx, E), axis=-1, keepdims=True)
    oh1 = eidx == idx1
    masked = jnp.where(oh1, -jnp.inf, logits)
    m2 = jnp.max(masked, axis=-1, keepdims=True)
    idx2 = jnp.min(jnp.where(masked == m2, eidx, E), axis=-1, keepdims=True)
    oh2 = eidx == idx2

    ex = jnp.exp(m2 - m1)
    den = 1.0 + ex
    g1v = 1.0 / den
    g2v = ex / den
    gates = jnp.where(oh1, g1v, jnp.where(oh2, g2v, 0.0))
    gates_ref[...] = gates

    @pl.when(bt == 0)
    def _():
        run_ref[...] = jnp.zeros_like(run_ref)
        imp_ref[...] = jnp.zeros_like(imp_ref)

    mask = (oh1 | oh2).astype(jnp.float32)
    bi = lax.broadcasted_iota(jnp.int32, (TB, TB), 0)
    bj = lax.broadcasted_iota(jnp.int32, (TB, TB), 1)
    tri = (bj < bi).astype(jnp.bfloat16)
    pos = lax.dot_general(tri, mask.astype(jnp.bfloat16),
                          (((1,), (0,)), ((), ())),
                          preferred_element_type=jnp.float32)
    posg = pos + run_ref[...]

    pos1v = jnp.sum(jnp.where(oh1, posg, 0.0), axis=-1).astype(jnp.int32)
    pos2v = jnp.sum(jnp.where(oh2, posg, 0.0), axis=-1).astype(jnp.int32)
    e0_ref[...] = idx1
    e1_ref[...] = idx2
    pos0_ref[...] = pos1v[:, None]
    pos1_ref[...] = pos2v[:, None]
    g0_ref[...] = g1v
    g1_ref[...] = g2v

    run_ref[...] = run_ref[...] + jnp.sum(mask, axis=0, keepdims=True)
    imp_ref[...] = imp_ref[...] + jnp.sum(gates, axis=0, keepdims=True)

    @pl.when(bt == NBT - 1)
    def _():
        cntf = run_ref[...]
        ntiles = jnp.floor((cntf + (TMOE - 1)) * (1.0 / TMOE))
        tri8a = lax.broadcasted_iota(jnp.int32, (E, E), 0)
        tri8b = lax.broadcasted_iota(jnp.int32, (E, E), 1)
        incl = (tri8a <= tri8b).astype(jnp.bfloat16)
        ends = lax.dot_general(ntiles.astype(jnp.bfloat16), incl,
                               (((1,), (0,)), ((), ())),
                               preferred_element_type=jnp.float32)
        ends_i = ends.astype(jnp.int32)
        ntiles_i = ntiles.astype(jnp.int32)
        base_t = ends_i - ntiles_i
        brow = base_t * SROWS
        base_ref[...] = jnp.concatenate(
            [brow, jnp.zeros((1, E), jnp.int32)], axis=1)
        ti = lax.broadcasted_iota(jnp.int32, (NTMAX, E), 0)
        texp = jnp.sum((jnp.broadcast_to(ends_i, (NTMAX, E)) <= ti)
                       .astype(jnp.int32), axis=1, keepdims=True)
        texp_ref[...] = jnp.minimum(texp, E - 1)
        nta_ref[...] = ends_i[:, E - 1:E]
        imp = imp_ref[...]
        mu = jnp.mean(imp)
        var = jnp.mean((imp - mu) ** 2)
        bloss_ref[...] = jnp.reshape(0.01 * var / (mu * mu + 1e-10), (1, 1))


def _k1(mod0, mod1, mod2, p0_w, p0_b, p2_w, p2_b, ln_g, ln_b, gate_w, gate_b):
    full = lambda s: pl.BlockSpec(s, lambda bt: (0,) * len(s))
    outs = (
        jax.ShapeDtypeStruct((B, M, D), jnp.float32),
        jax.ShapeDtypeStruct((B, M, D), jnp.float32),
        jax.ShapeDtypeStruct((B, E), jnp.float32),
        jax.ShapeDtypeStruct((B, 1), jnp.int32),
        jax.ShapeDtypeStruct((B, 1), jnp.int32),
        jax.ShapeDtypeStruct((B, 1), jnp.int32),
        jax.ShapeDtypeStruct((B, 1), jnp.int32),
        jax.ShapeDtypeStruct((B, 1), jnp.float32),
        jax.ShapeDtypeStruct((B, 1), jnp.float32),
        jax.ShapeDtypeStruct((1, 2 * E), jnp.int32),
        jax.ShapeDtypeStruct((NTMAX, 1), jnp.int32),
        jax.ShapeDtypeStruct((1, 1), jnp.int32),
        jax.ShapeDtypeStruct((1, 1), jnp.float32),
    )
    return pl.pallas_call(
        _k1_body,
        grid=(NBT,),
        in_specs=[
            pl.BlockSpec((TB, 768), lambda bt: (bt, 0)),
            pl.BlockSpec((TB, D), lambda bt: (bt, 0)),
            pl.BlockSpec((TB, 512), lambda bt: (bt, 0)),
            full((D, 768)),
            full((1, D)),
            full((D, 512)),
            full((1, D)),
            full((1, D)),
            full((1, D)),
            full((E, M * D)),
            full((1, E)),
        ],
        out_specs=(
            pl.BlockSpec((TB, M, D), lambda bt: (bt, 0, 0)),
            pl.BlockSpec((TB, M, D), lambda bt: (bt, 0, 0)),
            pl.BlockSpec((TB, E), lambda bt: (bt, 0)),
            pl.BlockSpec((TB, 1), lambda bt: (bt, 0)),
            pl.BlockSpec((TB, 1), lambda bt: (bt, 0)),
            pl.BlockSpec((TB, 1), lambda bt: (bt, 0)),
            pl.BlockSpec((TB, 1), lambda bt: (bt, 0)),
            pl.BlockSpec((TB, 1), lambda bt: (bt, 0)),
            pl.BlockSpec((TB, 1), lambda bt: (bt, 0)),
            full((1, 2 * E)),
            full((NTMAX, 1)),
            full((1, 1)),
            full((1, 1)),
        ),
        out_shape=outs,
        scratch_shapes=[
            pltpu.VMEM((1, E), jnp.float32),
            pltpu.VMEM((1, E), jnp.float32),
        ],
    )(mod0, mod1, mod2, p0_w, p0_b.reshape(1, D), p2_w, p2_b.reshape(1, D),
      ln_g.reshape(1, D), ln_b.reshape(1, D), gate_w, gate_b.reshape(1, E))


def _build_idx(idx_ref, se_ref, sp_ref, bvm_ref):
    it = lax.broadcasted_iota(jnp.int32, (16,), 0)
    for g in range(SPW // 16):
        ev = se_ref[pl.ds(g * 16, 16)]
        pv = sp_ref[pl.ds(g * 16, 16)]
        bv = plsc.load_gather(bvm_ref, [ev])
        rowbase = bv + pv * 3
        for r in range(M):
            plsc.store_scatter(idx_ref, [it * 3 + (g * 48 + r)], rowbase + r)


def _sc_scatter_body(tok_hbm, e0_hbm, e1_hbm, p0_hbm, p1_hbm, base_hbm,
                     disp_hbm, buf, idx, se, sp, bvm, sem):
    wid = lax.axis_index("s") * NC + lax.axis_index("c")
    base = wid * SPW
    pltpu.sync_copy(tok_hbm.at[pl.ds(base * M, WROWS)], buf)
    pltpu.sync_copy(base_hbm, bvm)
    pltpu.sync_copy(e0_hbm.at[pl.ds(base, SPW)], se)
    pltpu.sync_copy(p0_hbm.at[pl.ds(base, SPW)], sp)
    _build_idx(idx, se, sp, bvm)
    pltpu.async_copy(buf, disp_hbm.at[idx], sem).wait()
    pltpu.sync_copy(e1_hbm.at[pl.ds(base, SPW)], se)
    pltpu.sync_copy(p1_hbm.at[pl.ds(base, SPW)], sp)
    _build_idx(idx, se, sp, bvm)
    pltpu.async_copy(buf, disp_hbm.at[idx], sem).wait()


def _sc_scatter(tokens_flat, e0, e1, p0, p1, base_row):
    mesh = plsc.VectorSubcoreMesh(core_axis_name="c", subcore_axis_name="s")
    return pl.kernel(
        _sc_scatter_body,
        out_type=jax.ShapeDtypeStruct((NDROWS, D), jnp.float32),
        mesh=mesh,
        compiler_params=pltpu.CompilerParams(needs_layout_passes=False),
        scratch_types=[
            pltpu.VMEM((WROWS, D), jnp.float32),
            pltpu.VMEM((WROWS,), jnp.int32),
            pltpu.VMEM((SPW,), jnp.int32),
            pltpu.VMEM((SPW,), jnp.int32),
            pltpu.VMEM((2 * E,), jnp.int32),
            pltpu.SemaphoreType.DMA,
        ],
    )(tokens_flat, e0, e1, p0, p1, base_row)


def _sc_gather_body(y_hbm, e0_hbm, e1_hbm, p0_hbm, p1_hbm, base_hbm,
                    y0_hbm, y1_hbm, buf, idx, se, sp, bvm, sem):
    wid = lax.axis_index("s") * NC + lax.axis_index("c")
    base = wid * SPW
    pltpu.sync_copy(base_hbm, bvm)
    pltpu.sync_copy(e0_hbm.at[pl.ds(base, SPW)], se)
    pltpu.sync_copy(p0_hbm.at[pl.ds(base, SPW)], sp)
    _build_idx(idx, se, sp, bvm)
    pltpu.async_copy(y_hbm.at[idx], buf, sem).wait()
    pltpu.sync_copy(buf, y0_hbm.at[pl.ds(base * M, WROWS)])
    pltpu.sync_copy(e1_hbm.at[pl.ds(base, SPW)], se)
    pltpu.sync_copy(p1_hbm.at[pl.ds(base, SPW)], sp)
    _build_idx(idx, se, sp, bvm)
    pltpu.async_copy(y_hbm.at[idx], buf, sem).wait()
    pltpu.sync_copy(buf, y1_hbm.at[pl.ds(base * M, WROWS)])


def _sc_gather(y, e0, e1, p0, p1, base_row):
    mesh = plsc.VectorSubcoreMesh(core_axis_name="c", subcore_axis_name="s")
    return pl.kernel(
        _sc_gather_body,
        out_type=(jax.ShapeDtypeStruct((B * M, D), jnp.float32),
                  jax.ShapeDtypeStruct((B * M, D), jnp.float32)),
        mesh=mesh,
        compiler_params=pltpu.CompilerParams(needs_layout_passes=False),
        scratch_types=[
            pltpu.VMEM((WROWS, D), jnp.float32),
            pltpu.VMEM((WROWS,), jnp.int32),
            pltpu.VMEM((SPW,), jnp.int32),
            pltpu.VMEM((SPW,), jnp.int32),
            pltpu.VMEM((2 * E,), jnp.int32),
            pltpu.SemaphoreType.DMA,
        ],
    )(y, e0, e1, p0, p1, base_row)


def _moe_body(texp_ref, nta_ref, x_ref, w1_ref, b1_ref, w2_ref, b2_ref,
              y_ref):
    t = pl.program_id(0)

    @pl.when(t < nta_ref[0])
    def _():
        x = x_ref[...].astype(jnp.bfloat16)
        w1 = w1_ref[0].astype(jnp.bfloat16)
        h = jnp.dot(x, w1, preferred_element_type=jnp.float32) + b1_ref[0]
        hb = h.astype(jnp.bfloat16)
        u = hb * jnp.bfloat16(0.7978845608) * (
            jnp.bfloat16(1.0) + jnp.bfloat16(0.044715) * hb * hb)
        g = jnp.bfloat16(0.5) * hb * (jnp.bfloat16(1.0) + jnp.tanh(u))
        w2 = w2_ref[0].astype(jnp.bfloat16)
        y_ref[...] = (jnp.dot(g, w2,
                              preferred_element_type=jnp.float32) + b2_ref[0])


def _sparse_moe(texp, nta, disp, e_w1, e_b1, e_w2, e_b2):
    grid_spec = pltpu.PrefetchScalarGridSpec(
        num_scalar_prefetch=2,
        grid=(NTMAX,),
        in_specs=[
            pl.BlockSpec((SROWS, D), lambda t, te, na: (t, 0)),
            pl.BlockSpec((1, D, H), lambda t, te, na: (te[t], 0, 0)),
            pl.BlockSpec((1, 1, H), lambda t, te, na: (te[t], 0, 0)),
            pl.BlockSpec((1, H, D), lambda t, te, na: (te[t], 0, 0)),
            pl.BlockSpec((1, 1, D), lambda t, te, na: (te[t], 0, 0)),
        ],
        out_specs=pl.BlockSpec(
            (SROWS, D), lambda t, te, na: (jnp.where(t < na[0], t, NTMAX), 0)),
    )
    return pl.pallas_call(
        _moe_body,
        grid_spec=grid_spec,
        out_shape=jax.ShapeDtypeStruct(((NTMAX + 1) * SROWS, D), jnp.float32),
    )(texp, nta, disp, e_w1, e_b1.reshape(E, 1, H), e_w2,
      e_b2.reshape(E, 1, D))


def _comb_body(y0_ref, y1_ref, g0_ref, g1_ref, out_ref, pooled_ref):
    gw0 = g0_ref[...][:, :, None]
    gw1 = g1_ref[...][:, :, None]
    o = gw0 * y0_ref[...] + gw1 * y1_ref[...]
    out_ref[...] = o
    pooled_ref[...] = (o[:, 0, :] + o[:, 1, :] + o[:, 2, :]) * (1.0 / M)


def _combine(y0, y1, g0, g1):
    outs = (
        jax.ShapeDtypeStruct((B, M, D), jnp.float32),
        jax.ShapeDtypeStruct((B, D), jnp.float32),
    )
    return pl.pallas_call(
        _comb_body,
        grid=(NBT,),
        in_specs=[
            pl.BlockSpec((TB, M, D), lambda bt: (bt, 0, 0)),
            pl.BlockSpec((TB, M, D), lambda bt: (bt, 0, 0)),
            pl.BlockSpec((TB, 1), lambda bt: (bt, 0)),
            pl.BlockSpec((TB, 1), lambda bt: (bt, 0)),
        ],
        out_specs=(
            pl.BlockSpec((TB, M, D), lambda bt: (bt, 0, 0)),
            pl.BlockSpec((TB, D), lambda bt: (bt, 0)),
        ),
        out_shape=outs,
    )(y0, y1, g0, g1)


def kernel(mod0, mod1, mod2, p0_w, p0_b, p2_w, p2_b, ln_g, ln_b, gate_w,
           gate_b, e_w1, e_b1, e_w2, e_b2):
    (projected, tokens, gates, e0, e1, pos0, pos1, g0, g1, base_row, texp,
     nta, bloss) = _k1(mod0, mod1, mod2, p0_w, p0_b, p2_w, p2_b, ln_g, ln_b,
                       gate_w, gate_b)
    tokens_flat = tokens.reshape(B * M, D)
    e0f = e0.reshape(B)
    e1f = e1.reshape(B)
    p0f = pos0.reshape(B)
    p1f = pos1.reshape(B)
    basef = base_row.reshape(2 * E)
    disp = _sc_scatter(tokens_flat, e0f, e1f, p0f, p1f, basef)
    y = _sparse_moe(texp.reshape(NTMAX), nta.reshape(1), disp,
                    e_w1, e_b1, e_w2, e_b2)
    y0, y1 = _sc_gather(y, e0f, e1f, p0f, p1f, basef)
    modality_tokens, pooled = _combine(y0.reshape(B, M, D),
                                       y1.reshape(B, M, D), g0, g1)
    return pooled, modality_tokens, projected, gates, bloss[0, 0]


# speedup vs baseline: 2.9843x; 1.0231x over previous
import jax
import jax.numpy as jnp
from jax import lax
from jax.experimental import pallas as pl
from jax.experimental.pallas import tpu as pltpu
from jax.experimental.pallas import tpu_sc as plsc

B = 1024
D = 1024
M = 3
E = 8
H = 2048
TB = 256
NBT = B // TB
TMOE = 256
NTMAX = (2 * B) // TMOE + E
SROWS = TMOE * M
NDROWS = NTMAX * SROWS

NC = 2
NS = 16
NW = NC * NS
SPW = B // NW
WROWS = SPW * M


def _erf(x):
    a1, a2, a3, a4, a5 = (
        0.254829592, -0.284496736, 1.421413741, -1.453152027, 1.061405429)
    p = 0.3275911
    s = jnp.sign(x)
    ax = jnp.abs(x)
    t = 1.0 / (1.0 + p * ax)
    poly = t * (a1 + t * (a2 + t * (a3 + t * (a4 + t * a5))))
    y = 1.0 - poly * jnp.exp(-ax * ax)
    return s * y


def _gelu(x):
    return 0.5 * x * (1.0 + _erf(x * 0.7071067811865476))


def _k1_body(mod0_ref, mod1_ref, mod2_ref, p0w_ref, p0b_ref, p2w_ref,
             p2b_ref, lng_ref, lnb_ref, gw_ref, gb_ref,
             proj_ref, tok_ref, gates_ref, e0_ref, e1_ref, pos0_ref,
             pos1_ref, g0_ref, g1_ref, base_ref, texp_ref, nta_ref,
             bloss_ref, run_ref, imp_ref):
    bt = pl.program_id(0)
    bf = jnp.bfloat16

    def bdot(a, b):
        return lax.dot_general(a.astype(bf), b.astype(bf),
                               (((1,), (1,)), ((), ())),
                               preferred_element_type=jnp.float32)

    t0 = bdot(mod0_ref[...], p0w_ref[...]) + p0b_ref[...]
    t1 = mod1_ref[...]
    t2 = bdot(mod2_ref[...], p2w_ref[...]) + p2b_ref[...]

    proj_ref[:, 0, :] = t0
    proj_ref[:, 1, :] = t1
    proj_ref[:, 2, :] = t2

    def ln(t):
        mu = jnp.mean(t, axis=-1, keepdims=True)
        var = jnp.mean((t - mu) ** 2, axis=-1, keepdims=True)
        return (t - mu) / jnp.sqrt(var + 1e-5) * lng_ref[...] + lnb_ref[...]

    n0, n1, n2 = ln(t0), ln(t1), ln(t2)
    tok_ref[:, 0, :] = n0
    tok_ref[:, 1, :] = n1
    tok_ref[:, 2, :] = n2

    ctx = jnp.concatenate([n0, n1, n2], axis=-1)
    logits = bdot(ctx, gw_ref[...]) + gb_ref[...]

    eidx = lax.broadcasted_iota(jnp.int32, (TB, E), 1)
    m1 = jnp.max(logits, axis=-1, keepdims=True)
    idx1 = jnp.min(jnp.where(logits == m1, eidx, E), axis=-1, keepdims=True)
    oh1 = eidx == idx1
    masked = jnp.where(oh1, -jnp.inf, logits)
    m2 = jnp.max(masked, axis=-1, keepdims=True)
    idx2 = jnp.min(jnp.where(masked == m2, eidx, E), axis=-1, keepdims=True)
    oh2 = eidx == idx2

    ex = jnp.exp(m2 - m1)
    den = 1.0 + ex
    g1v = 1.0 / den
    g2v = ex / den
    gates = jnp.where(oh1, g1v, jnp.where(oh2, g2v, 0.0))
    gates_ref[...] = gates

    @pl.when(bt == 0)
    def _():
        run_ref[...] = jnp.zeros_like(run_ref)
        imp_ref[...] = jnp.zeros_like(imp_ref)

    mask = (oh1 | oh2).astype(jnp.float32)
    bi = lax.broadcasted_iota(jnp.int32, (TB, TB), 0)
    bj = lax.broadcasted_iota(jnp.int32, (TB, TB), 1)
    tri = (bj < bi).astype(jnp.bfloat16)
    pos = lax.dot_general(tri, mask.astype(jnp.bfloat16),
                          (((1,), (0,)), ((), ())),
                          preferred_element_type=jnp.float32)
    posg = pos + run_ref[...]

    pos1v = jnp.sum(jnp.where(oh1, posg, 0.0), axis=-1).astype(jnp.int32)
    pos2v = jnp.sum(jnp.where(oh2, posg, 0.0), axis=-1).astype(jnp.int32)
    e0_ref[...] = idx1
    e1_ref[...] = idx2
    pos0_ref[...] = pos1v[:, None]
    pos1_ref[...] = pos2v[:, None]
    g0_ref[...] = g1v
    g1_ref[...] = g2v

    run_ref[...] = run_ref[...] + jnp.sum(mask, axis=0, keepdims=True)
    imp_ref[...] = imp_ref[...] + jnp.sum(gates, axis=0, keepdims=True)

    @pl.when(bt == NBT - 1)
    def _():
        cntf = run_ref[...]
        ntiles = jnp.floor((cntf + (TMOE - 1)) * (1.0 / TMOE))
        tri8a = lax.broadcasted_iota(jnp.int32, (E, E), 0)
        tri8b = lax.broadcasted_iota(jnp.int32, (E, E), 1)
        incl = (tri8a <= tri8b).astype(jnp.bfloat16)
        ends = lax.dot_general(ntiles.astype(jnp.bfloat16), incl,
                               (((1,), (0,)), ((), ())),
                               preferred_element_type=jnp.float32)
        ends_i = ends.astype(jnp.int32)
        ntiles_i = ntiles.astype(jnp.int32)
        base_t = ends_i - ntiles_i
        brow = base_t * SROWS
        base_ref[...] = jnp.concatenate(
            [brow, jnp.zeros((1, E), jnp.int32)], axis=1)
        ti = lax.broadcasted_iota(jnp.int32, (NTMAX, E), 0)
        texp = jnp.sum((jnp.broadcast_to(ends_i, (NTMAX, E)) <= ti)
                       .astype(jnp.int32), axis=1, keepdims=True)
        texp_ref[...] = jnp.minimum(texp, E - 1)
        nta_ref[...] = ends_i[:, E - 1:E]
        imp = imp_ref[...]
        mu = jnp.mean(imp)
        var = jnp.mean((imp - mu) ** 2)
        bloss_ref[...] = jnp.reshape(0.01 * var / (mu * mu + 1e-10), (1, 1))


def _k1(mod0, mod1, mod2, p0_w, p0_b, p2_w, p2_b, ln_g, ln_b, gate_w, gate_b):
    full = lambda s: pl.BlockSpec(s, lambda bt: (0,) * len(s))
    outs = (
        jax.ShapeDtypeStruct((B, M, D), jnp.float32),
        jax.ShapeDtypeStruct((B, M, D), jnp.float32),
        jax.ShapeDtypeStruct((B, E), jnp.float32),
        jax.ShapeDtypeStruct((B, 1), jnp.int32),
        jax.ShapeDtypeStruct((B, 1), jnp.int32),
        jax.ShapeDtypeStruct((B, 1), jnp.int32),
        jax.ShapeDtypeStruct((B, 1), jnp.int32),
        jax.ShapeDtypeStruct((B, 1), jnp.float32),
        jax.ShapeDtypeStruct((B, 1), jnp.float32),
        jax.ShapeDtypeStruct((1, 2 * E), jnp.int32),
        jax.ShapeDtypeStruct((NTMAX, 1), jnp.int32),
        jax.ShapeDtypeStruct((1, 1), jnp.int32),
        jax.ShapeDtypeStruct((1, 1), jnp.float32),
    )
    return pl.pallas_call(
        _k1_body,
        grid=(NBT,),
        in_specs=[
            pl.BlockSpec((TB, 768), lambda bt: (bt, 0)),
            pl.BlockSpec((TB, D), lambda bt: (bt, 0)),
            pl.BlockSpec((TB, 512), lambda bt: (bt, 0)),
            full((D, 768)),
            full((1, D)),
            full((D, 512)),
            full((1, D)),
            full((1, D)),
            full((1, D)),
            full((E, M * D)),
            full((1, E)),
        ],
        out_specs=(
            pl.BlockSpec((TB, M, D), lambda bt: (bt, 0, 0)),
            pl.BlockSpec((TB, M, D), lambda bt: (bt, 0, 0)),
            pl.BlockSpec((TB, E), lambda bt: (bt, 0)),
            pl.BlockSpec((TB, 1), lambda bt: (bt, 0)),
            pl.BlockSpec((TB, 1), lambda bt: (bt, 0)),
            pl.BlockSpec((TB, 1), lambda bt: (bt, 0)),
            pl.BlockSpec((TB, 1), lambda bt: (bt, 0)),
            pl.BlockSpec((TB, 1), lambda bt: (bt, 0)),
            pl.BlockSpec((TB, 1), lambda bt: (bt, 0)),
            full((1, 2 * E)),
            full((NTMAX, 1)),
            full((1, 1)),
            full((1, 1)),
        ),
        out_shape=outs,
        scratch_shapes=[
            pltpu.VMEM((1, E), jnp.float32),
            pltpu.VMEM((1, E), jnp.float32),
        ],
    )(mod0, mod1, mod2, p0_w, p0_b.reshape(1, D), p2_w, p2_b.reshape(1, D),
      ln_g.reshape(1, D), ln_b.reshape(1, D), gate_w, gate_b.reshape(1, E))


def _build_idx(idx_ref, se_ref, sp_ref, bvm_ref):
    it = lax.broadcasted_iota(jnp.int32, (16,), 0)
    for g in range(SPW // 16):
        ev = se_ref[pl.ds(g * 16, 16)]
        pv = sp_ref[pl.ds(g * 16, 16)]
        bv = plsc.load_gather(bvm_ref, [ev])
        rowbase = bv + pv * 3
        for r in range(M):
            plsc.store_scatter(idx_ref, [it * 3 + (g * 48 + r)], rowbase + r)


def _sc_scatter_body(tok_hbm, e0_hbm, e1_hbm, p0_hbm, p1_hbm, base_hbm,
                     disp_hbm, buf, idx, se, sp, bvm, sem):
    wid = lax.axis_index("s") * NC + lax.axis_index("c")
    base = wid * SPW
    pltpu.sync_copy(tok_hbm.at[pl.ds(base * M, WROWS)], buf)
    pltpu.sync_copy(base_hbm, bvm)
    pltpu.sync_copy(e0_hbm.at[pl.ds(base, SPW)], se)
    pltpu.sync_copy(p0_hbm.at[pl.ds(base, SPW)], sp)
    _build_idx(idx, se, sp, bvm)
    pltpu.async_copy(buf, disp_hbm.at[idx], sem).wait()
    pltpu.sync_copy(e1_hbm.at[pl.ds(base, SPW)], se)
    pltpu.sync_copy(p1_hbm.at[pl.ds(base, SPW)], sp)
    _build_idx(idx, se, sp, bvm)
    pltpu.async_copy(buf, disp_hbm.at[idx], sem).wait()


def _sc_scatter(tokens_flat, e0, e1, p0, p1, base_row):
    mesh = plsc.VectorSubcoreMesh(core_axis_name="c", subcore_axis_name="s")
    return pl.kernel(
        _sc_scatter_body,
        out_type=jax.ShapeDtypeStruct((NDROWS, D), jnp.float32),
        mesh=mesh,
        compiler_params=pltpu.CompilerParams(needs_layout_passes=False),
        scratch_types=[
            pltpu.VMEM((WROWS, D), jnp.float32),
            pltpu.VMEM((WROWS,), jnp.int32),
            pltpu.VMEM((SPW,), jnp.int32),
            pltpu.VMEM((SPW,), jnp.int32),
            pltpu.VMEM((2 * E,), jnp.int32),
            pltpu.SemaphoreType.DMA,
        ],
    )(tokens_flat, e0, e1, p0, p1, base_row)


def _sc_gather_body(y_hbm, e0_hbm, e1_hbm, p0_hbm, p1_hbm, base_hbm,
                    y0_hbm, y1_hbm, buf, idx, se, sp, bvm, sem):
    wid = lax.axis_index("s") * NC + lax.axis_index("c")
    base = wid * SPW
    pltpu.sync_copy(base_hbm, bvm)
    pltpu.sync_copy(e0_hbm.at[pl.ds(base, SPW)], se)
    pltpu.sync_copy(p0_hbm.at[pl.ds(base, SPW)], sp)
    _build_idx(idx, se, sp, bvm)
    pltpu.async_copy(y_hbm.at[idx], buf, sem).wait()
    pltpu.sync_copy(buf, y0_hbm.at[pl.ds(base * M, WROWS)])
    pltpu.sync_copy(e1_hbm.at[pl.ds(base, SPW)], se)
    pltpu.sync_copy(p1_hbm.at[pl.ds(base, SPW)], sp)
    _build_idx(idx, se, sp, bvm)
    pltpu.async_copy(y_hbm.at[idx], buf, sem).wait()
    pltpu.sync_copy(buf, y1_hbm.at[pl.ds(base * M, WROWS)])


def _sc_gather(y, e0, e1, p0, p1, base_row):
    mesh = plsc.VectorSubcoreMesh(core_axis_name="c", subcore_axis_name="s")
    return pl.kernel(
        _sc_gather_body,
        out_type=(jax.ShapeDtypeStruct((B * M, D), jnp.float32),
                  jax.ShapeDtypeStruct((B * M, D), jnp.float32)),
        mesh=mesh,
        compiler_params=pltpu.CompilerParams(needs_layout_passes=False),
        scratch_types=[
            pltpu.VMEM((WROWS, D), jnp.float32),
            pltpu.VMEM((WROWS,), jnp.int32),
            pltpu.VMEM((SPW,), jnp.int32),
            pltpu.VMEM((SPW,), jnp.int32),
            pltpu.VMEM((2 * E,), jnp.int32),
            pltpu.SemaphoreType.DMA,
        ],
    )(y, e0, e1, p0, p1, base_row)


def _moe_body(texp_ref, nta_ref, x_ref, w1_ref, b1_ref, w2_ref, b2_ref,
              y_ref):
    t = pl.program_id(0)

    @pl.when(t < nta_ref[0])
    def _():
        x = x_ref[...].astype(jnp.bfloat16)
        w1 = w1_ref[0].astype(jnp.bfloat16)
        h = jnp.dot(x, w1, preferred_element_type=jnp.float32) + b1_ref[0]
        hb = h.astype(jnp.bfloat16)
        u = hb * jnp.bfloat16(0.7978845608) * (
            jnp.bfloat16(1.0) + jnp.bfloat16(0.044715) * hb * hb)
        g = jnp.bfloat16(0.5) * hb * (jnp.bfloat16(1.0) + jnp.tanh(u))
        w2 = w2_ref[0].astype(jnp.bfloat16)
        y_ref[...] = (jnp.dot(g, w2,
                              preferred_element_type=jnp.float32) + b2_ref[0])


def _sparse_moe(texp, nta, disp, e_w1, e_b1, e_w2, e_b2):
    grid_spec = pltpu.PrefetchScalarGridSpec(
        num_scalar_prefetch=2,
        grid=(NTMAX,),
        in_specs=[
            pl.BlockSpec((SROWS, D), lambda t, te, na: (t, 0)),
            pl.BlockSpec((1, D, H), lambda t, te, na: (te[t], 0, 0)),
            pl.BlockSpec((1, 1, H), lambda t, te, na: (te[t], 0, 0)),
            pl.BlockSpec((1, H, D), lambda t, te, na: (te[t], 0, 0)),
            pl.BlockSpec((1, 1, D), lambda t, te, na: (te[t], 0, 0)),
        ],
        out_specs=pl.BlockSpec(
            (SROWS, D), lambda t, te, na: (jnp.where(t < na[0], t, NTMAX), 0)),
    )
    return pl.pallas_call(
        _moe_body,
        grid_spec=grid_spec,
        out_shape=jax.ShapeDtypeStruct(((NTMAX + 1) * SROWS, D), jnp.float32),
    )(texp, nta, disp, e_w1, e_b1.reshape(E, 1, H), e_w2,
      e_b2.reshape(E, 1, D))


def _comb_body(y0_ref, y1_ref, g0_ref, g1_ref, out_ref, pooled_ref):
    gw0 = g0_ref[...][:, :, None]
    gw1 = g1_ref[...][:, :, None]
    o = gw0 * y0_ref[...] + gw1 * y1_ref[...]
    out_ref[...] = o
    pooled_ref[...] = (o[:, 0, :] + o[:, 1, :] + o[:, 2, :]) * (1.0 / M)


def _combine(y0, y1, g0, g1):
    outs = (
        jax.ShapeDtypeStruct((B, M, D), jnp.float32),
        jax.ShapeDtypeStruct((B, D), jnp.float32),
    )
    return pl.pallas_call(
        _comb_body,
        grid=(NBT,),
        in_specs=[
            pl.BlockSpec((TB, M, D), lambda bt: (bt, 0, 0)),
            pl.BlockSpec((TB, M, D), lambda bt: (bt, 0, 0)),
            pl.BlockSpec((TB, 1), lambda bt: (bt, 0)),
            pl.BlockSpec((TB, 1), lambda bt: (bt, 0)),
        ],
        out_specs=(
            pl.BlockSpec((TB, M, D), lambda bt: (bt, 0, 0)),
            pl.BlockSpec((TB, D), lambda bt: (bt, 0)),
        ),
        out_shape=outs,
    )(y0, y1, g0, g1)


def kernel(mod0, mod1, mod2, p0_w, p0_b, p2_w, p2_b, ln_g, ln_b, gate_w,
           gate_b, e_w1, e_b1, e_w2, e_b2):
    (projected, tokens, gates, e0, e1, pos0, pos1, g0, g1, base_row, texp,
     nta, bloss) = _k1(mod0, mod1, mod2, p0_w, p0_b, p2_w, p2_b, ln_g, ln_b,
                       gate_w, gate_b)
    tokens_flat = tokens.reshape(B * M, D)
    e0f = e0.reshape(B)
    e1f = e1.reshape(B)
    p0f = pos0.reshape(B)
    p1f = pos1.reshape(B)
    basef = base_row.reshape(2 * E)
    disp = _sc_scatter(tokens_flat, e0f, e1f, p0f, p1f, basef)
    y = _sparse_moe(texp.reshape(NTMAX), nta.reshape(1), disp,
                    e_w1, e_b1, e_w2, e_b2)
    y0, y1 = _sc_gather(y, e0f, e1f, p0f, p1f, basef)
    modality_tokens, pooled = _combine(y0.reshape(B, M, D),
                                       y1.reshape(B, M, D), g0, g1)
    return pooled, modality_tokens, projected, gates, bloss[0, 0]


# speedup vs baseline: 3.0204x; 1.0121x over previous
import jax
import jax.numpy as jnp
from jax import lax
from jax.experimental import pallas as pl
from jax.experimental.pallas import tpu as pltpu
from jax.experimental.pallas import tpu_sc as plsc

B = 1024
D = 1024
M = 3
E = 8
H = 2048
TB = 256
NBT = B // TB
TMOE = 256
NTMAX = (2 * B) // TMOE + E
SROWS = TMOE * M
NDROWS = NTMAX * SROWS

NC = 2
NS = 16
NW = NC * NS
SPW = B // NW
WROWS = SPW * M


def _erf(x):
    a1, a2, a3, a4, a5 = (
        0.254829592, -0.284496736, 1.421413741, -1.453152027, 1.061405429)
    p = 0.3275911
    s = jnp.sign(x)
    ax = jnp.abs(x)
    t = 1.0 / (1.0 + p * ax)
    poly = t * (a1 + t * (a2 + t * (a3 + t * (a4 + t * a5))))
    y = 1.0 - poly * jnp.exp(-ax * ax)
    return s * y


def _gelu(x):
    return 0.5 * x * (1.0 + _erf(x * 0.7071067811865476))


def _k1_body(mod0_ref, mod1_ref, mod2_ref, p0w_ref, p0b_ref, p2w_ref,
             p2b_ref, lng_ref, lnb_ref, gw_ref, gb_ref,
             proj_ref, tok_ref, gates_ref, e0_ref, e1_ref, pos0_ref,
             pos1_ref, g0_ref, g1_ref, base_ref, texp_ref, nta_ref,
             bloss_ref, run_ref, imp_ref):
    bt = pl.program_id(0)
    bf = jnp.bfloat16

    def bdot(a, b):
        return lax.dot_general(a.astype(bf), b.astype(bf),
                               (((1,), (1,)), ((), ())),
                               preferred_element_type=jnp.float32)

    t0 = bdot(mod0_ref[...], p0w_ref[...]) + p0b_ref[...]
    t1 = mod1_ref[...]
    t2 = bdot(mod2_ref[...], p2w_ref[...]) + p2b_ref[...]

    proj_ref[:, 0, :] = t0
    proj_ref[:, 1, :] = t1
    proj_ref[:, 2, :] = t2

    def ln(t):
        mu = jnp.mean(t, axis=-1, keepdims=True)
        var = jnp.mean((t - mu) ** 2, axis=-1, keepdims=True)
        return (t - mu) / jnp.sqrt(var + 1e-5) * lng_ref[...] + lnb_ref[...]

    n0, n1, n2 = ln(t0), ln(t1), ln(t2)
    tok_ref[:, 0, :] = n0
    tok_ref[:, 1, :] = n1
    tok_ref[:, 2, :] = n2

    ctx = jnp.concatenate([n0, n1, n2], axis=-1)
    logits = bdot(ctx, gw_ref[...]) + gb_ref[...]

    eidx = lax.broadcasted_iota(jnp.int32, (TB, E), 1)
    m1 = jnp.max(logits, axis=-1, keepdims=True)
    idx1 = jnp.min(jnp.where(logits == m1, eidx, E), axis=-1, keepdims=True)
    oh1 = eidx == idx1
    masked = jnp.where(oh1, -jnp.inf, logits)
    m2 = jnp.max(masked, axis=-1, keepdims=True)
    idx2 = jnp.min(jnp.where(masked == m2, eidx, E), axis=-1, keepdims=True)
    oh2 = eidx == idx2

    ex = jnp.exp(m2 - m1)
    den = 1.0 + ex
    g1v = 1.0 / den
    g2v = ex / den
    gates = jnp.where(oh1, g1v, jnp.where(oh2, g2v, 0.0))
    gates_ref[...] = gates

    @pl.when(bt == 0)
    def _():
        run_ref[...] = jnp.zeros_like(run_ref)
        imp_ref[...] = jnp.zeros_like(imp_ref)

    mask = (oh1 | oh2).astype(jnp.float32)
    bi = lax.broadcasted_iota(jnp.int32, (TB, TB), 0)
    bj = lax.broadcasted_iota(jnp.int32, (TB, TB), 1)
    tri = (bj < bi).astype(jnp.bfloat16)
    pos = lax.dot_general(tri, mask.astype(jnp.bfloat16),
                          (((1,), (0,)), ((), ())),
                          preferred_element_type=jnp.float32)
    posg = pos + run_ref[...]

    pos1v = jnp.sum(jnp.where(oh1, posg, 0.0), axis=-1).astype(jnp.int32)
    pos2v = jnp.sum(jnp.where(oh2, posg, 0.0), axis=-1).astype(jnp.int32)
    e0_ref[...] = idx1
    e1_ref[...] = idx2
    pos0_ref[...] = pos1v[:, None]
    pos1_ref[...] = pos2v[:, None]
    g0_ref[...] = g1v
    g1_ref[...] = g2v

    run_ref[...] = run_ref[...] + jnp.sum(mask, axis=0, keepdims=True)
    imp_ref[...] = imp_ref[...] + jnp.sum(gates, axis=0, keepdims=True)

    @pl.when(bt == NBT - 1)
    def _():
        cntf = run_ref[...]
        ntiles = jnp.floor((cntf + (TMOE - 1)) * (1.0 / TMOE))
        tri8a = lax.broadcasted_iota(jnp.int32, (E, E), 0)
        tri8b = lax.broadcasted_iota(jnp.int32, (E, E), 1)
        incl = (tri8a <= tri8b).astype(jnp.bfloat16)
        ends = lax.dot_general(ntiles.astype(jnp.bfloat16), incl,
                               (((1,), (0,)), ((), ())),
                               preferred_element_type=jnp.float32)
        ends_i = ends.astype(jnp.int32)
        ntiles_i = ntiles.astype(jnp.int32)
        base_t = ends_i - ntiles_i
        brow = base_t * SROWS
        base_ref[...] = jnp.concatenate(
            [brow, jnp.zeros((1, E), jnp.int32)], axis=1)
        ti = lax.broadcasted_iota(jnp.int32, (NTMAX, E), 0)
        texp = jnp.sum((jnp.broadcast_to(ends_i, (NTMAX, E)) <= ti)
                       .astype(jnp.int32), axis=1, keepdims=True)
        texp_ref[...] = jnp.minimum(texp, E - 1)
        nta_ref[...] = ends_i[:, E - 1:E]
        imp = imp_ref[...]
        mu = jnp.mean(imp)
        var = jnp.mean((imp - mu) ** 2)
        bloss_ref[...] = jnp.reshape(0.01 * var / (mu * mu + 1e-10), (1, 1))


def _k1(mod0, mod1, mod2, p0_w, p0_b, p2_w, p2_b, ln_g, ln_b, gate_w, gate_b):
    full = lambda s: pl.BlockSpec(s, lambda bt: (0,) * len(s))
    outs = (
        jax.ShapeDtypeStruct((B, M, D), jnp.float32),
        jax.ShapeDtypeStruct((B, M, D), jnp.float32),
        jax.ShapeDtypeStruct((B, E), jnp.float32),
        jax.ShapeDtypeStruct((B, 1), jnp.int32),
        jax.ShapeDtypeStruct((B, 1), jnp.int32),
        jax.ShapeDtypeStruct((B, 1), jnp.int32),
        jax.ShapeDtypeStruct((B, 1), jnp.int32),
        jax.ShapeDtypeStruct((B, 1), jnp.float32),
        jax.ShapeDtypeStruct((B, 1), jnp.float32),
        jax.ShapeDtypeStruct((1, 2 * E), jnp.int32),
        jax.ShapeDtypeStruct((NTMAX, 1), jnp.int32),
        jax.ShapeDtypeStruct((1, 1), jnp.int32),
        jax.ShapeDtypeStruct((1, 1), jnp.float32),
    )
    return pl.pallas_call(
        _k1_body,
        grid=(NBT,),
        in_specs=[
            pl.BlockSpec((TB, 768), lambda bt: (bt, 0)),
            pl.BlockSpec((TB, D), lambda bt: (bt, 0)),
            pl.BlockSpec((TB, 512), lambda bt: (bt, 0)),
            full((D, 768)),
            full((1, D)),
            full((D, 512)),
            full((1, D)),
            full((1, D)),
            full((1, D)),
            full((E, M * D)),
            full((1, E)),
        ],
        out_specs=(
            pl.BlockSpec((TB, M, D), lambda bt: (bt, 0, 0)),
            pl.BlockSpec((TB, M, D), lambda bt: (bt, 0, 0)),
            pl.BlockSpec((TB, E), lambda bt: (bt, 0)),
            pl.BlockSpec((TB, 1), lambda bt: (bt, 0)),
            pl.BlockSpec((TB, 1), lambda bt: (bt, 0)),
            pl.BlockSpec((TB, 1), lambda bt: (bt, 0)),
            pl.BlockSpec((TB, 1), lambda bt: (bt, 0)),
            pl.BlockSpec((TB, 1), lambda bt: (bt, 0)),
            pl.BlockSpec((TB, 1), lambda bt: (bt, 0)),
            full((1, 2 * E)),
            full((NTMAX, 1)),
            full((1, 1)),
            full((1, 1)),
        ),
        out_shape=outs,
        scratch_shapes=[
            pltpu.VMEM((1, E), jnp.float32),
            pltpu.VMEM((1, E), jnp.float32),
        ],
    )(mod0, mod1, mod2, p0_w, p0_b.reshape(1, D), p2_w, p2_b.reshape(1, D),
      ln_g.reshape(1, D), ln_b.reshape(1, D), gate_w, gate_b.reshape(1, E))


def _build_idx(idx_ref, se_ref, sp_ref, bvm_ref):
    it = lax.broadcasted_iota(jnp.int32, (16,), 0)
    for g in range(SPW // 16):
        ev = se_ref[pl.ds(g * 16, 16)]
        pv = sp_ref[pl.ds(g * 16, 16)]
        bv = plsc.load_gather(bvm_ref, [ev])
        rowbase = bv + pv * 3
        for r in range(M):
            plsc.store_scatter(idx_ref, [it * 3 + (g * 48 + r)], rowbase + r)


def _sc_scatter_body(tok_hbm, e0_hbm, e1_hbm, p0_hbm, p1_hbm, base_hbm,
                     disp_hbm, buf, idx0, idx1, se, sp, bvm, sem, semt):
    wid = lax.axis_index("s") * NC + lax.axis_index("c")
    base = wid * SPW
    tok_cp = pltpu.async_copy(tok_hbm.at[pl.ds(base * M, WROWS)], buf, semt)
    pltpu.sync_copy(base_hbm, bvm)
    pltpu.sync_copy(e0_hbm.at[pl.ds(base, SPW)], se)
    pltpu.sync_copy(p0_hbm.at[pl.ds(base, SPW)], sp)
    _build_idx(idx0, se, sp, bvm)
    pltpu.sync_copy(e1_hbm.at[pl.ds(base, SPW)], se)
    pltpu.sync_copy(p1_hbm.at[pl.ds(base, SPW)], sp)
    _build_idx(idx1, se, sp, bvm)
    tok_cp.wait()
    c0 = pltpu.async_copy(buf, disp_hbm.at[idx0], sem)
    c1 = pltpu.async_copy(buf, disp_hbm.at[idx1], sem)
    c0.wait()
    c1.wait()


def _sc_scatter(tokens_flat, e0, e1, p0, p1, base_row):
    mesh = plsc.VectorSubcoreMesh(core_axis_name="c", subcore_axis_name="s")
    return pl.kernel(
        _sc_scatter_body,
        out_type=jax.ShapeDtypeStruct((NDROWS, D), jnp.float32),
        mesh=mesh,
        compiler_params=pltpu.CompilerParams(needs_layout_passes=False),
        scratch_types=[
            pltpu.VMEM((WROWS, D), jnp.float32),
            pltpu.VMEM((WROWS,), jnp.int32),
            pltpu.VMEM((WROWS,), jnp.int32),
            pltpu.VMEM((SPW,), jnp.int32),
            pltpu.VMEM((SPW,), jnp.int32),
            pltpu.VMEM((2 * E,), jnp.int32),
            pltpu.SemaphoreType.DMA,
            pltpu.SemaphoreType.DMA,
        ],
    )(tokens_flat, e0, e1, p0, p1, base_row)


def _sc_gather_body(y_hbm, e0_hbm, e1_hbm, p0_hbm, p1_hbm, base_hbm,
                    y0_hbm, y1_hbm, buf0, idx0, idx1, se, sp, bvm,
                    sem0, sem1):
    wid = lax.axis_index("s") * NC + lax.axis_index("c")
    base = wid * SPW
    pltpu.sync_copy(base_hbm, bvm)
    pltpu.sync_copy(e0_hbm.at[pl.ds(base, SPW)], se)
    pltpu.sync_copy(p0_hbm.at[pl.ds(base, SPW)], sp)
    _build_idx(idx0, se, sp, bvm)
    g0 = pltpu.async_copy(y_hbm.at[idx0], buf0, sem0)
    pltpu.sync_copy(e1_hbm.at[pl.ds(base, SPW)], se)
    pltpu.sync_copy(p1_hbm.at[pl.ds(base, SPW)], sp)
    _build_idx(idx1, se, sp, bvm)
    g0.wait()
    pltpu.sync_copy(buf0, y0_hbm.at[pl.ds(base * M, WROWS)])
    pltpu.async_copy(y_hbm.at[idx1], buf0, sem1).wait()
    pltpu.sync_copy(buf0, y1_hbm.at[pl.ds(base * M, WROWS)])


def _sc_gather(y, e0, e1, p0, p1, base_row):
    mesh = plsc.VectorSubcoreMesh(core_axis_name="c", subcore_axis_name="s")
    return pl.kernel(
        _sc_gather_body,
        out_type=(jax.ShapeDtypeStruct((B * M, D), jnp.float32),
                  jax.ShapeDtypeStruct((B * M, D), jnp.float32)),
        mesh=mesh,
        compiler_params=pltpu.CompilerParams(needs_layout_passes=False),
        scratch_types=[
            pltpu.VMEM((WROWS, D), jnp.float32),
            pltpu.VMEM((WROWS,), jnp.int32),
            pltpu.VMEM((WROWS,), jnp.int32),
            pltpu.VMEM((SPW,), jnp.int32),
            pltpu.VMEM((SPW,), jnp.int32),
            pltpu.VMEM((2 * E,), jnp.int32),
            pltpu.SemaphoreType.DMA,
            pltpu.SemaphoreType.DMA,
        ],
    )(y, e0, e1, p0, p1, base_row)


def _moe_body(texp_ref, nta_ref, x_ref, w1_ref, b1_ref, w2_ref, b2_ref,
              y_ref):
    t = pl.program_id(0)

    @pl.when(t < nta_ref[0])
    def _():
        x = x_ref[...].astype(jnp.bfloat16)
        w1 = w1_ref[0].astype(jnp.bfloat16)
        h = jnp.dot(x, w1, preferred_element_type=jnp.float32) + b1_ref[0]
        hb = h.astype(jnp.bfloat16)
        u = hb * jnp.bfloat16(0.7978845608) * (
            jnp.bfloat16(1.0) + jnp.bfloat16(0.044715) * hb * hb)
        g = jnp.bfloat16(0.5) * hb * (jnp.bfloat16(1.0) + jnp.tanh(u))
        w2 = w2_ref[0].astype(jnp.bfloat16)
        y_ref[...] = (jnp.dot(g, w2,
                              preferred_element_type=jnp.float32) + b2_ref[0])


def _sparse_moe(texp, nta, disp, e_w1, e_b1, e_w2, e_b2):
    grid_spec = pltpu.PrefetchScalarGridSpec(
        num_scalar_prefetch=2,
        grid=(NTMAX,),
        in_specs=[
            pl.BlockSpec((SROWS, D), lambda t, te, na: (t, 0)),
            pl.BlockSpec((1, D, H), lambda t, te, na: (te[t], 0, 0)),
            pl.BlockSpec((1, 1, H), lambda t, te, na: (te[t], 0, 0)),
            pl.BlockSpec((1, H, D), lambda t, te, na: (te[t], 0, 0)),
            pl.BlockSpec((1, 1, D), lambda t, te, na: (te[t], 0, 0)),
        ],
        out_specs=pl.BlockSpec(
            (SROWS, D), lambda t, te, na: (jnp.where(t < na[0], t, NTMAX), 0)),
    )
    return pl.pallas_call(
        _moe_body,
        grid_spec=grid_spec,
        out_shape=jax.ShapeDtypeStruct(((NTMAX + 1) * SROWS, D), jnp.float32),
    )(texp, nta, disp, e_w1, e_b1.reshape(E, 1, H), e_w2,
      e_b2.reshape(E, 1, D))


def _comb_body(y0_ref, y1_ref, g0_ref, g1_ref, out_ref, pooled_ref):
    gw0 = g0_ref[...][:, :, None]
    gw1 = g1_ref[...][:, :, None]
    o = gw0 * y0_ref[...] + gw1 * y1_ref[...]
    out_ref[...] = o
    pooled_ref[...] = (o[:, 0, :] + o[:, 1, :] + o[:, 2, :]) * (1.0 / M)


def _combine(y0, y1, g0, g1):
    outs = (
        jax.ShapeDtypeStruct((B, M, D), jnp.float32),
        jax.ShapeDtypeStruct((B, D), jnp.float32),
    )
    return pl.pallas_call(
        _comb_body,
        grid=(NBT,),
        in_specs=[
            pl.BlockSpec((TB, M, D), lambda bt: (bt, 0, 0)),
            pl.BlockSpec((TB, M, D), lambda bt: (bt, 0, 0)),
            pl.BlockSpec((TB, 1), lambda bt: (bt, 0)),
            pl.BlockSpec((TB, 1), lambda bt: (bt, 0)),
        ],
        out_specs=(
            pl.BlockSpec((TB, M, D), lambda bt: (bt, 0, 0)),
            pl.BlockSpec((TB, D), lambda bt: (bt, 0)),
        ),
        out_shape=outs,
    )(y0, y1, g0, g1)


def kernel(mod0, mod1, mod2, p0_w, p0_b, p2_w, p2_b, ln_g, ln_b, gate_w,
           gate_b, e_w1, e_b1, e_w2, e_b2):
    (projected, tokens, gates, e0, e1, pos0, pos1, g0, g1, base_row, texp,
     nta, bloss) = _k1(mod0, mod1, mod2, p0_w, p0_b, p2_w, p2_b, ln_g, ln_b,
                       gate_w, gate_b)
    tokens_flat = tokens.reshape(B * M, D)
    e0f = e0.reshape(B)
    e1f = e1.reshape(B)
    p0f = pos0.reshape(B)
    p1f = pos1.reshape(B)
    basef = base_row.reshape(2 * E)
    disp = _sc_scatter(tokens_flat, e0f, e1f, p0f, p1f, basef)
    y = _sparse_moe(texp.reshape(NTMAX), nta.reshape(1), disp,
                    e_w1, e_b1, e_w2, e_b2)
    y0, y1 = _sc_gather(y, e0f, e1f, p0f, p1f, basef)
    modality_tokens, pooled = _combine(y0.reshape(B, M, D),
                                       y1.reshape(B, M, D), g0, g1)
    return pooled, modality_tokens, projected, gates, bloss[0, 0]
